# Initial kernel scaffold; baseline (speedup 1.0000x reference)
import functools

import jax
import jax.numpy as jnp
from jax import lax
from jax.experimental import pallas as pl
from jax.experimental.pallas import tpu as pltpu
from jax.experimental.pallas import tpu_sc as plsc

N = 10000
E = 320000
D = 128
H = 64
B = 8
MAXN = 1250

NC, NS = 2, 16
NW = NC * NS
NP = 10240
ROWS_PER_TILE = NP // NS
CH = 128
EP = 327680
EDGES_PER_W = EP // NW
NCHUNK = EDGES_PER_W // CH

_mesh = plsc.VectorSubcoreMesh(core_axis_name="c", subcore_axis_name="s")


@functools.partial(
    pl.kernel,
    out_type=jax.ShapeDtypeStruct((NC, NP), jnp.float32),
    mesh=_mesh,
    scratch_types=[
        pltpu.VMEM_SHARED((NP,), jnp.float32),
        pltpu.VMEM((CH,), jnp.int32),
        pltpu.VMEM((CH,), jnp.float32),
        pltpu.VMEM((ROWS_PER_TILE,), jnp.float32),
    ],
)
def _sc_deg(dst_hbm, zrow_hbm, out_hbm, acc_sh, dst_v, ones_v, zrow_v):
    c = lax.axis_index("c")
    s = lax.axis_index("s")
    base = (s * NC + c) * EDGES_PER_W
    rbase = s * ROWS_PER_TILE
    pltpu.sync_copy(zrow_hbm, zrow_v)
    pltpu.sync_copy(zrow_v, acc_sh.at[pl.ds(rbase, ROWS_PER_TILE)])
    for i in range(CH // 16):
        ones_v[pl.ds(i * 16, 16)] = jnp.ones((16,), jnp.float32)
    plsc.subcore_barrier()

    def chunk(i, carry):
        pltpu.sync_copy(dst_hbm.at[pl.ds(base + i * CH, CH)], dst_v)
        pltpu.sync_copy(ones_v, acc_sh.at[dst_v], add=True)
        return carry

    lax.fori_loop(0, NCHUNK, chunk, 0)
    plsc.subcore_barrier()
    pltpu.sync_copy(acc_sh.at[pl.ds(rbase, ROWS_PER_TILE)],
                    out_hbm.at[c, pl.ds(rbase, ROWS_PER_TILE)])


def _make_sc_segsum(width):
    @functools.partial(
        pl.kernel,
        out_type=jax.ShapeDtypeStruct((NC, NP, width), jnp.float32),
        mesh=_mesh,
        scratch_types=[
            pltpu.VMEM_SHARED((NP, width), jnp.float32),
            pltpu.VMEM((CH,), jnp.int32),
            pltpu.VMEM((CH,), jnp.int32),
            pltpu.VMEM((CH, width), jnp.float32),
            pltpu.SemaphoreType.DMA,
        ],
    )
    def _sc_segsum(g_hbm, src_hbm, dst_hbm, ztile_hbm, out_hbm,
                   acc_sh, src_v, dst_v, rows_v, gsem):
        c = lax.axis_index("c")
        s = lax.axis_index("s")
        base = (s * NC + c) * EDGES_PER_W
        rbase = s * ROWS_PER_TILE
        pltpu.sync_copy(ztile_hbm, rows_v)
        for r in range(ROWS_PER_TILE // CH):
            pltpu.sync_copy(rows_v, acc_sh.at[pl.ds(rbase + r * CH, CH)])
        plsc.subcore_barrier()

        def chunk(i, carry):
            pltpu.sync_copy(src_hbm.at[pl.ds(base + i * CH, CH)], src_v)
            pltpu.sync_copy(dst_hbm.at[pl.ds(base + i * CH, CH)], dst_v)
            pltpu.async_copy(g_hbm.at[src_v], rows_v, gsem).wait()
            pltpu.sync_copy(rows_v, acc_sh.at[dst_v], add=True)
            return carry

        lax.fori_loop(0, NCHUNK, chunk, 0)
        plsc.subcore_barrier()
        pltpu.sync_copy(acc_sh.at[pl.ds(rbase, ROWS_PER_TILE)],
                        out_hbm.at[c, pl.ds(rbase, ROWS_PER_TILE)])

    return _sc_segsum


_sc_segsum64 = _make_sc_segsum(H)


LW = 16
EDGES_PER_T3 = EP // NS
NCHUNK3 = EDGES_PER_T3 // CH


@functools.partial(
    pl.kernel,
    out_type=jax.ShapeDtypeStruct((NP, LW), jnp.float32),
    mesh=_mesh,
    scratch_types=[
        pltpu.VMEM_SHARED((NP, LW), jnp.float32),
        pltpu.VMEM((CH,), jnp.int32),
        pltpu.VMEM((CH,), jnp.int32),
        pltpu.VMEM((CH, LW), jnp.float32),
        pltpu.VMEM((ROWS_PER_TILE, LW), jnp.float32),
        pltpu.VMEM((ROWS_PER_TILE, LW), jnp.float32),
        pltpu.VMEM((ROWS_PER_TILE,), jnp.float32),
        pltpu.VMEM((ROWS_PER_TILE,), jnp.int32),
        pltpu.VMEM((CH,), jnp.int32),
        pltpu.SemaphoreType.DMA,
    ],
)
def _sc_leaf(g3_hbm, src_hbm, dst_hbm, sb3_hbm, norm_hbm, fidx_hbm, ztile_hbm,
             out_hbm, acc_sh, src_v, dst_v, rows_v, a3_v, sb_v, norm_v,
             fidx_v, fc_v, gsem):
    c = lax.axis_index("c")
    s = lax.axis_index("s")

    @pl.when(c == 0)
    def _():
        base = s * EDGES_PER_T3
        rbase = s * ROWS_PER_TILE
        pltpu.sync_copy(ztile_hbm, rows_v)
        for r in range(ROWS_PER_TILE // CH):
            pltpu.sync_copy(rows_v, acc_sh.at[pl.ds(rbase + r * CH, CH)])
        plsc.subcore_barrier()

        def chunk(i, carry):
            pltpu.sync_copy(src_hbm.at[pl.ds(base + i * CH, CH)], src_v)
            pltpu.sync_copy(dst_hbm.at[pl.ds(base + i * CH, CH)], dst_v)
            pltpu.async_copy(g3_hbm.at[src_v], rows_v, gsem).wait()
            pltpu.sync_copy(rows_v, acc_sh.at[dst_v], add=True)
            return carry

        lax.fori_loop(0, NCHUNK3, chunk, 0)
        plsc.subcore_barrier()

        pltpu.sync_copy(acc_sh.at[pl.ds(rbase, ROWS_PER_TILE)], a3_v)
        pltpu.sync_copy(sb3_hbm.at[pl.ds(rbase, ROWS_PER_TILE)], sb_v)
        pltpu.sync_copy(norm_hbm.at[pl.ds(rbase, ROWS_PER_TILE)], norm_v)
        pltpu.sync_copy(fidx_hbm.at[pl.ds(rbase, ROWS_PER_TILE)], fidx_v)
        for gi in range(ROWS_PER_TILE // 16):
            rid = lax.iota(jnp.int32, 16) + gi * 16
            nv = norm_v[pl.ds(gi * 16, 16)]
            for j in range(2):
                jv = jnp.full((16,), j, jnp.int32)
                av = plsc.load_gather(a3_v, [rid, jv])
                sv = plsc.load_gather(sb_v, [rid, jv])
                plsc.store_scatter(a3_v, [rid, jv], nv * av + sv)
        for ci in range(ROWS_PER_TILE // CH):
            pltpu.sync_copy(fidx_v.at[pl.ds(ci * CH, CH)], fc_v)
            pltpu.async_copy(a3_v.at[pl.ds(ci * CH, CH)],
                             out_hbm.at[fc_v], gsem).wait()


R = 1024


def _tc1_body(d_ref, x_ref, w_ref, h1_ref, g1_ref, n_ref):
    deg = d_ref[:, 0:1] + d_ref[:, 1:2] + 1.0
    norm = lax.rsqrt(deg)
    h1 = jnp.dot(x_ref[...], w_ref[...], preferred_element_type=jnp.float32)
    h1_ref[...] = h1
    g1_ref[...] = norm * h1
    n_ref[...] = norm


def _tc2_body(a_ref, h1_ref, n_ref, b_ref, w_ref, t2_ref, g2_ref):
    a = a_ref[0] + a_ref[1]
    n = n_ref[...]
    pre = n * a + (n * n) * h1_ref[...] + b_ref[...]
    t2 = jnp.dot(pre, w_ref[...], preferred_element_type=jnp.float32)
    t2_ref[...] = t2
    g2_ref[...] = n * t2


def _tc3_body(a_ref, t2_ref, n_ref, bb_ref, wl_ref, bl_ref, we_ref, be_ref,
              g3_ref, sb3_ref, ye_ref):
    i = pl.program_id(0)
    a = a_ref[0] + a_ref[1]
    n = n_ref[...]
    pre = n * a + (n * n) * t2_ref[...] + bb_ref[...]
    h2 = jnp.where(pre >= 0, pre, 0.01 * pre)
    t3 = jnp.dot(h2, wl_ref[...], preferred_element_type=jnp.float32)
    g3_ref[...] = n * t3
    sb3_ref[...] = (n * n) * t3 + bl_ref[...]
    v = jnp.dot(h2, we_ref[...], preferred_element_type=jnp.float32)
    rows = lax.broadcasted_iota(jnp.int32, (R, 1), 0) + i * R
    gid = rows // MAXN
    valid = rows < N

    @pl.when(i == 0)
    def _():
        ye_ref[...] = jnp.broadcast_to(be_ref[0, 0], (1, B))

    parts = [jnp.sum(jnp.where((gid == gg) & valid, v, 0.0))
             for gg in range(B)]
    ye_ref[...] = ye_ref[...] + jnp.stack(parts).reshape(1, B)


def _row_spec(width):
    return pl.BlockSpec((R, width), lambda i: (i, 0))


def _part_spec(width):
    return pl.BlockSpec((NC, R, width), lambda i: (0, i, 0))


def _full_spec(shape):
    return pl.BlockSpec(shape, lambda i: tuple(0 for _ in shape))


_tc1 = pl.pallas_call(
    _tc1_body,
    grid=(NP // R,),
    in_specs=[_row_spec(2), _row_spec(D), _full_spec((D, H))],
    out_specs=[_row_spec(H), _row_spec(H), _row_spec(1)],
    out_shape=[jax.ShapeDtypeStruct((NP, H), jnp.float32),
               jax.ShapeDtypeStruct((NP, H), jnp.float32),
               jax.ShapeDtypeStruct((NP, 1), jnp.float32)],
)

_tc2 = pl.pallas_call(
    _tc2_body,
    grid=(NP // R,),
    in_specs=[_part_spec(H), _row_spec(H), _row_spec(1),
              _full_spec((1, H)), _full_spec((H, H))],
    out_specs=[_row_spec(H), _row_spec(H)],
    out_shape=[jax.ShapeDtypeStruct((NP, H), jnp.float32),
               jax.ShapeDtypeStruct((NP, H), jnp.float32)],
)

_tc3 = pl.pallas_call(
    _tc3_body,
    grid=(NP // R,),
    in_specs=[_part_spec(H), _row_spec(H), _row_spec(1),
              _full_spec((1, H)), _full_spec((H, LW)), _full_spec((1, LW)),
              _full_spec((H, 1)), _full_spec((1, 1))],
    out_specs=[_row_spec(LW), _row_spec(LW),
               pl.BlockSpec((1, B), lambda i: (0, 0))],
    out_shape=[jax.ShapeDtypeStruct((NP, LW), jnp.float32),
               jax.ShapeDtypeStruct((NP, LW), jnp.float32),
               jax.ShapeDtypeStruct((1, B), jnp.float32)],
)


@jax.jit
def kernel(x, edge_index, k, batch, W_bb, b_bb, W_body, b_body,
           W_leaf, b_leaf, W_eos, b_eos):
    f32 = jnp.float32
    src = jnp.concatenate(
        [edge_index[0], jnp.full((EP - E,), N, jnp.int32)])
    dst = jnp.concatenate(
        [edge_index[1], jnp.full((EP - E,), N, jnp.int32)])
    flat_idx = batch.astype(jnp.int32) * MAXN + k.astype(jnp.int32)
    fidx = jnp.concatenate([flat_idx, jnp.full((NP - N,), N, jnp.int32)])
    x_p = jnp.pad(x, ((0, NP - N), (0, 0)))
    wl_p = jnp.pad(W_leaf, ((0, 0), (0, LW - 2)))
    bl_p = jnp.pad(b_leaf, ((0, LW - 2),)).reshape(1, LW)
    zrow = jnp.zeros((ROWS_PER_TILE,), f32)
    ztile64 = jnp.zeros((CH, H), f32)
    ztile16 = jnp.zeros((CH, LW), f32)

    degp = _sc_deg(dst, zrow)
    deg2 = jnp.transpose(degp)
    h1, g1, norm = _tc1(deg2, x_p, W_bb)
    a1 = _sc_segsum64(g1, src, dst, ztile64)
    t2, g2 = _tc2(a1, h1, norm, b_bb.reshape(1, H), W_body)
    a2 = _sc_segsum64(g2, src, dst, ztile64)
    g3, sb3, ye = _tc3(a2, t2, norm, b_body.reshape(1, H), wl_p, bl_p,
                       W_eos, b_eos.reshape(1, 1))
    y16 = _sc_leaf(g3, src, dst, sb3, norm.reshape(NP), fidx, ztile16)
    y_leaf = y16[:N, :2].reshape(B, MAXN * 2)
    y_eos = ye.reshape(B)
    return (y_leaf, y_eos)


# speedup vs baseline: 8.4908x; 8.4908x over previous
import functools

import jax
import jax.numpy as jnp
from jax import lax
from jax.experimental import pallas as pl
from jax.experimental.pallas import tpu as pltpu
from jax.experimental.pallas import tpu_sc as plsc

N = 10000
E = 320000
D = 128
H = 64
B = 8
MAXN = 1250

NC, NS = 2, 16
NW = NC * NS
NP = 10240
ROWS_PER_TILE = NP // NS
CH = 128
EP = 327680
EDGES_PER_W = EP // NW
NCHUNK = EDGES_PER_W // CH

_mesh = plsc.VectorSubcoreMesh(core_axis_name="c", subcore_axis_name="s")
_sc_params = pltpu.CompilerParams(use_tc_tiling_on_sc=False,
                                  needs_layout_passes=False)


@functools.partial(
    pl.kernel,
    out_type=jax.ShapeDtypeStruct((NC, NP), jnp.float32),
    mesh=_mesh,
    compiler_params=_sc_params,
    scratch_types=[
        pltpu.VMEM_SHARED((NP,), jnp.float32),
        pltpu.VMEM((CH,), jnp.int32),
        pltpu.VMEM((CH,), jnp.float32),
        pltpu.VMEM((ROWS_PER_TILE,), jnp.float32),
    ],
)
def _sc_deg(dst_hbm, zrow_hbm, out_hbm, acc_sh, dst_v, ones_v, zrow_v):
    c = lax.axis_index("c")
    s = lax.axis_index("s")
    base = (s * NC + c) * EDGES_PER_W
    rbase = s * ROWS_PER_TILE
    pltpu.sync_copy(zrow_hbm, zrow_v)
    pltpu.sync_copy(zrow_v, acc_sh.at[pl.ds(rbase, ROWS_PER_TILE)])
    for i in range(CH // 16):
        ones_v[pl.ds(i * 16, 16)] = jnp.ones((16,), jnp.float32)
    plsc.subcore_barrier()

    def chunk(i, carry):
        pltpu.sync_copy(dst_hbm.at[pl.ds(base + i * CH, CH)], dst_v)
        pltpu.sync_copy(ones_v, acc_sh.at[dst_v], add=True)
        return carry

    lax.fori_loop(0, NCHUNK, chunk, 0)
    plsc.subcore_barrier()
    pltpu.sync_copy(acc_sh.at[pl.ds(rbase, ROWS_PER_TILE)],
                    out_hbm.at[c, pl.ds(rbase, ROWS_PER_TILE)])


def _make_sc_segsum(width):
    @functools.partial(
        pl.kernel,
        out_type=jax.ShapeDtypeStruct((NC, NP, width), jnp.float32),
        mesh=_mesh,
        compiler_params=_sc_params,
        scratch_types=[
            pltpu.VMEM_SHARED((NP, width), jnp.float32),
            pltpu.VMEM((CH,), jnp.int32),
            pltpu.VMEM((CH,), jnp.int32),
            pltpu.VMEM((CH, width), jnp.float32),
            pltpu.SemaphoreType.DMA,
        ],
    )
    def _sc_segsum(g_hbm, src_hbm, dst_hbm, ztile_hbm, out_hbm,
                   acc_sh, src_v, dst_v, rows_v, gsem):
        c = lax.axis_index("c")
        s = lax.axis_index("s")
        base = (s * NC + c) * EDGES_PER_W
        rbase = s * ROWS_PER_TILE
        pltpu.sync_copy(ztile_hbm, rows_v)
        for r in range(ROWS_PER_TILE // CH):
            pltpu.sync_copy(rows_v, acc_sh.at[pl.ds(rbase + r * CH, CH)])
        plsc.subcore_barrier()

        def chunk(i, carry):
            pltpu.sync_copy(src_hbm.at[pl.ds(base + i * CH, CH)], src_v)
            pltpu.sync_copy(dst_hbm.at[pl.ds(base + i * CH, CH)], dst_v)
            pltpu.async_copy(g_hbm.at[src_v], rows_v, gsem).wait()
            pltpu.sync_copy(rows_v, acc_sh.at[dst_v], add=True)
            return carry

        lax.fori_loop(0, NCHUNK, chunk, 0)
        plsc.subcore_barrier()
        pltpu.sync_copy(acc_sh.at[pl.ds(rbase, ROWS_PER_TILE)],
                        out_hbm.at[c, pl.ds(rbase, ROWS_PER_TILE)])

    return _sc_segsum


_sc_segsum64 = _make_sc_segsum(H)


LW = 16
EDGES_PER_T3 = EP // NS
NCHUNK3 = EDGES_PER_T3 // CH


@functools.partial(
    pl.kernel,
    out_type=jax.ShapeDtypeStruct((NP, LW), jnp.float32),
    mesh=_mesh,
    compiler_params=_sc_params,
    scratch_types=[
        pltpu.VMEM_SHARED((NP, LW), jnp.float32),
        pltpu.VMEM((CH,), jnp.int32),
        pltpu.VMEM((CH,), jnp.int32),
        pltpu.VMEM((CH, LW), jnp.float32),
        pltpu.VMEM((ROWS_PER_TILE, LW), jnp.float32),
        pltpu.VMEM((ROWS_PER_TILE, LW), jnp.float32),
        pltpu.VMEM((ROWS_PER_TILE,), jnp.float32),
        pltpu.VMEM((CH,), jnp.int32),
        pltpu.SemaphoreType.DMA,
    ],
)
def _sc_leaf(g3_hbm, src_hbm, dst_hbm, sb3_hbm, norm_hbm, fidx_hbm, ztile_hbm,
             out_hbm, acc_sh, src_v, dst_v, rows_v, a3_v, sb_v, norm_v,
             fc_v, gsem):
    c = lax.axis_index("c")
    s = lax.axis_index("s")

    @pl.when(c == 0)
    def _():
        base = s * EDGES_PER_T3
        rbase = s * ROWS_PER_TILE
        pltpu.sync_copy(ztile_hbm, rows_v)
        for r in range(ROWS_PER_TILE // CH):
            pltpu.sync_copy(rows_v, acc_sh.at[pl.ds(rbase + r * CH, CH)])
        plsc.subcore_barrier()

        def chunk(i, carry):
            pltpu.sync_copy(src_hbm.at[pl.ds(base + i * CH, CH)], src_v)
            pltpu.sync_copy(dst_hbm.at[pl.ds(base + i * CH, CH)], dst_v)
            pltpu.async_copy(g3_hbm.at[src_v], rows_v, gsem).wait()
            pltpu.sync_copy(rows_v, acc_sh.at[dst_v], add=True)
            return carry

        lax.fori_loop(0, NCHUNK3, chunk, 0)
        plsc.subcore_barrier()

        pltpu.sync_copy(acc_sh.at[pl.ds(rbase, ROWS_PER_TILE)], a3_v)
        pltpu.sync_copy(sb3_hbm.at[pl.ds(rbase, ROWS_PER_TILE)], sb_v)
        pltpu.sync_copy(norm_hbm.at[pl.ds(rbase, ROWS_PER_TILE)], norm_v)
        for gi in range(ROWS_PER_TILE // 16):
            rid = lax.iota(jnp.int32, 16) + gi * 16
            nv = norm_v[pl.ds(gi * 16, 16)]
            for j in range(2):
                jv = jnp.full((16,), j, jnp.int32)
                av = plsc.load_gather(a3_v, [rid, jv])
                sv = plsc.load_gather(sb_v, [rid, jv])
                plsc.store_scatter(a3_v, [rid, jv], nv * av + sv)
        for ci in range(ROWS_PER_TILE // CH):
            pltpu.sync_copy(fidx_hbm.at[pl.ds(rbase + ci * CH, CH)], fc_v)
            pltpu.async_copy(a3_v.at[pl.ds(ci * CH, CH)],
                             out_hbm.at[fc_v], gsem).wait()


R = 1024


def _tc1_body(d_ref, x_ref, w_ref, h1_ref, g1_ref, n_ref):
    deg = d_ref[:, 0:1] + d_ref[:, 1:2] + 1.0
    norm = lax.rsqrt(deg)
    h1 = jnp.dot(x_ref[...], w_ref[...], preferred_element_type=jnp.float32)
    h1_ref[...] = h1
    g1_ref[...] = norm * h1
    n_ref[...] = norm


def _tc2_body(a_ref, h1_ref, n_ref, b_ref, w_ref, t2_ref, g2_ref):
    a = a_ref[0] + a_ref[1]
    n = n_ref[...]
    pre = n * a + (n * n) * h1_ref[...] + b_ref[...]
    t2 = jnp.dot(pre, w_ref[...], preferred_element_type=jnp.float32)
    t2_ref[...] = t2
    g2_ref[...] = n * t2


def _tc3_body(a_ref, t2_ref, n_ref, bb_ref, wl_ref, bl_ref, we_ref, be_ref,
              g3_ref, sb3_ref, ye_ref):
    i = pl.program_id(0)
    a = a_ref[0] + a_ref[1]
    n = n_ref[...]
    pre = n * a + (n * n) * t2_ref[...] + bb_ref[...]
    h2 = jnp.where(pre >= 0, pre, 0.01 * pre)
    t3 = jnp.dot(h2, wl_ref[...], preferred_element_type=jnp.float32)
    g3_ref[...] = n * t3
    sb3_ref[...] = (n * n) * t3 + bl_ref[...]
    v = jnp.dot(h2, we_ref[...], preferred_element_type=jnp.float32)
    rows = lax.broadcasted_iota(jnp.int32, (R, 1), 0) + i * R
    gid = rows // MAXN
    valid = rows < N

    @pl.when(i == 0)
    def _():
        ye_ref[...] = jnp.broadcast_to(be_ref[0, 0], (1, B))

    parts = [jnp.sum(jnp.where((gid == gg) & valid, v, 0.0))
             for gg in range(B)]
    ye_ref[...] = ye_ref[...] + jnp.stack(parts).reshape(1, B)


def _row_spec(width):
    return pl.BlockSpec((R, width), lambda i: (i, 0))


def _part_spec(width):
    return pl.BlockSpec((NC, R, width), lambda i: (0, i, 0))


def _full_spec(shape):
    return pl.BlockSpec(shape, lambda i: tuple(0 for _ in shape))


_tc1 = pl.pallas_call(
    _tc1_body,
    grid=(NP // R,),
    in_specs=[_row_spec(2), _row_spec(D), _full_spec((D, H))],
    out_specs=[_row_spec(H), _row_spec(H), _row_spec(1)],
    out_shape=[jax.ShapeDtypeStruct((NP, H), jnp.float32),
               jax.ShapeDtypeStruct((NP, H), jnp.float32),
               jax.ShapeDtypeStruct((NP, 1), jnp.float32)],
)

_tc2 = pl.pallas_call(
    _tc2_body,
    grid=(NP // R,),
    in_specs=[_part_spec(H), _row_spec(H), _row_spec(1),
              _full_spec((1, H)), _full_spec((H, H))],
    out_specs=[_row_spec(H), _row_spec(H)],
    out_shape=[jax.ShapeDtypeStruct((NP, H), jnp.float32),
               jax.ShapeDtypeStruct((NP, H), jnp.float32)],
)

_tc3 = pl.pallas_call(
    _tc3_body,
    grid=(NP // R,),
    in_specs=[_part_spec(H), _row_spec(H), _row_spec(1),
              _full_spec((1, H)), _full_spec((H, LW)), _full_spec((1, LW)),
              _full_spec((H, 1)), _full_spec((1, 1))],
    out_specs=[_row_spec(LW), _row_spec(LW),
               pl.BlockSpec((1, B), lambda i: (0, 0))],
    out_shape=[jax.ShapeDtypeStruct((NP, LW), jnp.float32),
               jax.ShapeDtypeStruct((NP, LW), jnp.float32),
               jax.ShapeDtypeStruct((1, B), jnp.float32)],
)


@jax.jit
def kernel(x, edge_index, k, batch, W_bb, b_bb, W_body, b_body,
           W_leaf, b_leaf, W_eos, b_eos):
    f32 = jnp.float32
    src = jnp.concatenate(
        [edge_index[0], jnp.full((EP - E,), N, jnp.int32)])
    dst = jnp.concatenate(
        [edge_index[1], jnp.full((EP - E,), N, jnp.int32)])
    flat_idx = batch.astype(jnp.int32) * MAXN + k.astype(jnp.int32)
    fidx = jnp.concatenate([flat_idx, jnp.full((NP - N,), N, jnp.int32)])
    x_p = jnp.pad(x, ((0, NP - N), (0, 0)))
    wl_p = jnp.pad(W_leaf, ((0, 0), (0, LW - 2)))
    bl_p = jnp.pad(b_leaf, ((0, LW - 2),)).reshape(1, LW)
    zrow = jnp.zeros((ROWS_PER_TILE,), f32)
    ztile64 = jnp.zeros((CH, H), f32)
    ztile16 = jnp.zeros((CH, LW), f32)

    degp = _sc_deg(dst, zrow)
    deg2 = jnp.transpose(degp)
    h1, g1, norm = _tc1(deg2, x_p, W_bb)
    a1 = _sc_segsum64(g1, src, dst, ztile64)
    t2, g2 = _tc2(a1, h1, norm, b_bb.reshape(1, H), W_body)
    a2 = _sc_segsum64(g2, src, dst, ztile64)
    g3, sb3, ye = _tc3(a2, t2, norm, b_body.reshape(1, H), wl_p, bl_p,
                       W_eos, b_eos.reshape(1, 1))
    y16 = _sc_leaf(g3, src, dst, sb3, norm.reshape(NP), fidx, ztile16)
    y_leaf = y16[:N, :2].reshape(B, MAXN * 2)
    y_eos = ye.reshape(B)
    return (y_leaf, y_eos)


# speedup vs baseline: 13.3528x; 1.5726x over previous
import functools

import jax
import jax.numpy as jnp
from jax import lax
from jax.experimental import pallas as pl
from jax.experimental.pallas import tpu as pltpu
from jax.experimental.pallas import tpu_sc as plsc

N = 10000
E = 320000
D = 128
H = 64
B = 8
MAXN = 1250

NC, NS = 2, 16
NW = NC * NS
NP = 10240
ROWS_PER_TILE = NP // NS
CH = 128
EP = 327680
EDGES_PER_W = EP // NW
NCHUNK = EDGES_PER_W // CH

_mesh = plsc.VectorSubcoreMesh(core_axis_name="c", subcore_axis_name="s")
_sc_params = pltpu.CompilerParams(use_tc_tiling_on_sc=False,
                                  needs_layout_passes=False)


@functools.partial(
    pl.kernel,
    out_type=jax.ShapeDtypeStruct((NC, NP), jnp.float32),
    mesh=_mesh,
    compiler_params=_sc_params,
    scratch_types=[
        pltpu.VMEM_SHARED((NP,), jnp.float32),
        pltpu.VMEM((NCHUNK, CH), jnp.int32),
        pltpu.VMEM((CH,), jnp.float32),
        pltpu.VMEM((ROWS_PER_TILE,), jnp.float32),
    ],
)
def _sc_deg(dst2_hbm, zrow_hbm, out_hbm, acc_sh, dstb_v, ones_v, zrow_v):
    c = lax.axis_index("c")
    s = lax.axis_index("s")
    wid = s * NC + c
    rbase = s * ROWS_PER_TILE
    pltpu.sync_copy(dst2_hbm.at[pl.ds(wid * NCHUNK, NCHUNK)], dstb_v)
    pltpu.sync_copy(zrow_hbm, zrow_v)
    pltpu.sync_copy(zrow_v, acc_sh.at[pl.ds(rbase, ROWS_PER_TILE)])
    for i in range(CH // 16):
        ones_v[pl.ds(i * 16, 16)] = jnp.ones((16,), jnp.float32)
    plsc.subcore_barrier()

    def chunk(i, carry):
        pltpu.sync_copy(ones_v, acc_sh.at[dstb_v.at[i]], add=True)
        return carry

    lax.fori_loop(0, NCHUNK, chunk, 0)
    plsc.subcore_barrier()
    pltpu.sync_copy(acc_sh.at[pl.ds(rbase, ROWS_PER_TILE)],
                    out_hbm.at[c, pl.ds(rbase, ROWS_PER_TILE)])


NBUF = 4


def _make_sc_segsum(width):
    @functools.partial(
        pl.kernel,
        out_type=jax.ShapeDtypeStruct((NC, NP, width), jnp.float32),
        mesh=_mesh,
        compiler_params=_sc_params,
        scratch_types=[
            pltpu.VMEM_SHARED((NP, width), jnp.float32),
            pltpu.VMEM((NCHUNK, CH), jnp.int32),
            pltpu.VMEM((NCHUNK, CH), jnp.int32),
            pltpu.VMEM((NBUF, CH, width), jnp.float32),
            pltpu.VMEM((CH, width), jnp.float32),
            pltpu.SemaphoreType.DMA,
            pltpu.SemaphoreType.DMA,
            pltpu.SemaphoreType.DMA,
            pltpu.SemaphoreType.DMA,
            pltpu.SemaphoreType.DMA,
            pltpu.SemaphoreType.DMA,
            pltpu.SemaphoreType.DMA,
            pltpu.SemaphoreType.DMA,
        ],
    )
    def _sc_segsum(g_hbm, src2_hbm, dst2_hbm, ztile_hbm, out_hbm,
                   acc_sh, srcb_v, dstb_v, bufs_v, zrows_v,
                   g0, g1, g2, g3, s0, s1, s2, s3):
        gsems = [g0, g1, g2, g3]
        ssems = [s0, s1, s2, s3]
        c = lax.axis_index("c")
        s = lax.axis_index("s")
        wid = s * NC + c
        rbase = s * ROWS_PER_TILE
        pltpu.sync_copy(src2_hbm.at[pl.ds(wid * NCHUNK, NCHUNK)], srcb_v)
        pltpu.sync_copy(dst2_hbm.at[pl.ds(wid * NCHUNK, NCHUNK)], dstb_v)
        pltpu.sync_copy(ztile_hbm, zrows_v)
        for r in range(ROWS_PER_TILE // CH):
            pltpu.sync_copy(zrows_v, acc_sh.at[pl.ds(rbase + r * CH, CH)])
        plsc.subcore_barrier()

        def sbody(q, carry):
            i0 = q * NBUF
            for b in range(NBUF):
                pltpu.async_copy(g_hbm.at[srcb_v.at[i0 + b]], bufs_v.at[b],
                                 gsems[b])
            for b in range(NBUF):
                pltpu.make_async_copy(g_hbm.at[srcb_v.at[i0 + b]],
                                      bufs_v.at[b], gsems[b]).wait()
                pltpu.async_copy(bufs_v.at[b], acc_sh.at[dstb_v.at[i0 + b]],
                                 ssems[b], add=True)
            for b in range(NBUF):
                pltpu.make_async_copy(bufs_v.at[b],
                                      acc_sh.at[dstb_v.at[i0 + b]],
                                      ssems[b]).wait()
            return carry

        lax.fori_loop(0, NCHUNK // NBUF, sbody, 0)
        plsc.subcore_barrier()
        pltpu.sync_copy(acc_sh.at[pl.ds(rbase, ROWS_PER_TILE)],
                        out_hbm.at[c, pl.ds(rbase, ROWS_PER_TILE)])

    return _sc_segsum


_sc_segsum64 = _make_sc_segsum(H)


LW = 16
EDGES_PER_T3 = EP // NS
NCHUNK3 = EDGES_PER_T3 // CH


@functools.partial(
    pl.kernel,
    out_type=jax.ShapeDtypeStruct((NP, LW), jnp.float32),
    mesh=_mesh,
    compiler_params=_sc_params,
    scratch_types=[
        pltpu.VMEM_SHARED((NP, LW), jnp.float32),
        pltpu.VMEM((NCHUNK3, CH), jnp.int32),
        pltpu.VMEM((NCHUNK3, CH), jnp.int32),
        pltpu.VMEM((NBUF, CH, LW), jnp.float32),
        pltpu.VMEM((ROWS_PER_TILE, LW), jnp.float32),
        pltpu.VMEM((ROWS_PER_TILE, LW), jnp.float32),
        pltpu.VMEM((ROWS_PER_TILE,), jnp.float32),
        pltpu.VMEM((CH,), jnp.int32),
        pltpu.SemaphoreType.DMA,
        pltpu.SemaphoreType.DMA,
        pltpu.SemaphoreType.DMA,
        pltpu.SemaphoreType.DMA,
        pltpu.SemaphoreType.DMA,
        pltpu.SemaphoreType.DMA,
        pltpu.SemaphoreType.DMA,
        pltpu.SemaphoreType.DMA,
    ],
)
def _sc_leaf(g3_hbm, src2_hbm, dst2_hbm, sb3_hbm, norm_hbm, fidx_hbm,
             ztile_hbm, out_hbm, acc_sh, srcb_v, dstb_v, bufs_v, a3_v, sb_v,
             norm_v, fc_v, g0, g1, g2, g3, s0, s1, s2, s3):
    gsems = [g0, g1, g2, g3]
    ssems = [s0, s1, s2, s3]
    c = lax.axis_index("c")
    s = lax.axis_index("s")

    @pl.when(c == 0)
    def _():
        rbase = s * ROWS_PER_TILE
        pltpu.sync_copy(src2_hbm.at[pl.ds(s * NCHUNK3, NCHUNK3)], srcb_v)
        pltpu.sync_copy(dst2_hbm.at[pl.ds(s * NCHUNK3, NCHUNK3)], dstb_v)
        pltpu.sync_copy(ztile_hbm, bufs_v.at[0])
        for r in range(ROWS_PER_TILE // CH):
            pltpu.sync_copy(bufs_v.at[0], acc_sh.at[pl.ds(rbase + r * CH, CH)])
        plsc.subcore_barrier()

        def sbody(q, carry):
            i0 = q * NBUF
            for b in range(NBUF):
                pltpu.async_copy(g3_hbm.at[srcb_v.at[i0 + b]], bufs_v.at[b],
                                 gsems[b])
            for b in range(NBUF):
                pltpu.make_async_copy(g3_hbm.at[srcb_v.at[i0 + b]],
                                      bufs_v.at[b], gsems[b]).wait()
                pltpu.async_copy(bufs_v.at[b], acc_sh.at[dstb_v.at[i0 + b]],
                                 ssems[b], add=True)
            for b in range(NBUF):
                pltpu.make_async_copy(bufs_v.at[b],
                                      acc_sh.at[dstb_v.at[i0 + b]],
                                      ssems[b]).wait()
            return carry

        lax.fori_loop(0, NCHUNK3 // NBUF, sbody, 0)
        plsc.subcore_barrier()

        pltpu.sync_copy(acc_sh.at[pl.ds(rbase, ROWS_PER_TILE)], a3_v)
        pltpu.sync_copy(sb3_hbm.at[pl.ds(rbase, ROWS_PER_TILE)], sb_v)
        pltpu.sync_copy(norm_hbm.at[pl.ds(rbase, ROWS_PER_TILE)], norm_v)
        for gi in range(ROWS_PER_TILE // 16):
            rid = lax.iota(jnp.int32, 16) + gi * 16
            nv = norm_v[pl.ds(gi * 16, 16)]
            for j in range(2):
                jv = jnp.full((16,), j, jnp.int32)
                av = plsc.load_gather(a3_v, [rid, jv])
                sv = plsc.load_gather(sb_v, [rid, jv])
                plsc.store_scatter(a3_v, [rid, jv], nv * av + sv)
        for ci in range(ROWS_PER_TILE // CH):
            pltpu.sync_copy(fidx_hbm.at[pl.ds(rbase + ci * CH, CH)], fc_v)
            pltpu.async_copy(a3_v.at[pl.ds(ci * CH, CH)],
                             out_hbm.at[fc_v], g0).wait()


R = 1024


def _tc1_body(d_ref, x_ref, w_ref, h1_ref, g1_ref, n_ref):
    deg = d_ref[:, 0:1] + d_ref[:, 1:2] + 1.0
    norm = lax.rsqrt(deg)
    h1 = jnp.dot(x_ref[...], w_ref[...], preferred_element_type=jnp.float32)
    h1_ref[...] = h1
    g1_ref[...] = norm * h1
    n_ref[...] = norm


def _tc2_body(a_ref, h1_ref, n_ref, b_ref, w_ref, t2_ref, g2_ref):
    a = a_ref[0] + a_ref[1]
    n = n_ref[...]
    pre = n * a + (n * n) * h1_ref[...] + b_ref[...]
    t2 = jnp.dot(pre, w_ref[...], preferred_element_type=jnp.float32)
    t2_ref[...] = t2
    g2_ref[...] = n * t2


def _tc3_body(a_ref, t2_ref, n_ref, bb_ref, wl_ref, bl_ref, we_ref, be_ref,
              g3_ref, sb3_ref, ye_ref):
    i = pl.program_id(0)
    a = a_ref[0] + a_ref[1]
    n = n_ref[...]
    pre = n * a + (n * n) * t2_ref[...] + bb_ref[...]
    h2 = jnp.where(pre >= 0, pre, 0.01 * pre)
    t3 = jnp.dot(h2, wl_ref[...], preferred_element_type=jnp.float32)
    g3_ref[...] = n * t3
    sb3_ref[...] = (n * n) * t3 + bl_ref[...]
    v = jnp.dot(h2, we_ref[...], preferred_element_type=jnp.float32)
    rows = lax.broadcasted_iota(jnp.int32, (R, 1), 0) + i * R
    gid = rows // MAXN
    valid = rows < N

    @pl.when(i == 0)
    def _():
        ye_ref[...] = jnp.broadcast_to(be_ref[0, 0], (1, B))

    parts = [jnp.sum(jnp.where((gid == gg) & valid, v, 0.0))
             for gg in range(B)]
    ye_ref[...] = ye_ref[...] + jnp.stack(parts).reshape(1, B)


def _row_spec(width):
    return pl.BlockSpec((R, width), lambda i: (i, 0))


def _part_spec(width):
    return pl.BlockSpec((NC, R, width), lambda i: (0, i, 0))


def _full_spec(shape):
    return pl.BlockSpec(shape, lambda i: tuple(0 for _ in shape))


_tc1 = pl.pallas_call(
    _tc1_body,
    grid=(NP // R,),
    in_specs=[_row_spec(2), _row_spec(D), _full_spec((D, H))],
    out_specs=[_row_spec(H), _row_spec(H), _row_spec(1)],
    out_shape=[jax.ShapeDtypeStruct((NP, H), jnp.float32),
               jax.ShapeDtypeStruct((NP, H), jnp.float32),
               jax.ShapeDtypeStruct((NP, 1), jnp.float32)],
)

_tc2 = pl.pallas_call(
    _tc2_body,
    grid=(NP // R,),
    in_specs=[_part_spec(H), _row_spec(H), _row_spec(1),
              _full_spec((1, H)), _full_spec((H, H))],
    out_specs=[_row_spec(H), _row_spec(H)],
    out_shape=[jax.ShapeDtypeStruct((NP, H), jnp.float32),
               jax.ShapeDtypeStruct((NP, H), jnp.float32)],
)

_tc3 = pl.pallas_call(
    _tc3_body,
    grid=(NP // R,),
    in_specs=[_part_spec(H), _row_spec(H), _row_spec(1),
              _full_spec((1, H)), _full_spec((H, LW)), _full_spec((1, LW)),
              _full_spec((H, 1)), _full_spec((1, 1))],
    out_specs=[_row_spec(LW), _row_spec(LW),
               pl.BlockSpec((1, B), lambda i: (0, 0))],
    out_shape=[jax.ShapeDtypeStruct((NP, LW), jnp.float32),
               jax.ShapeDtypeStruct((NP, LW), jnp.float32),
               jax.ShapeDtypeStruct((1, B), jnp.float32)],
)


@jax.jit
def kernel(x, edge_index, k, batch, W_bb, b_bb, W_body, b_body,
           W_leaf, b_leaf, W_eos, b_eos):
    f32 = jnp.float32
    src = jnp.concatenate(
        [edge_index[0], jnp.full((EP - E,), N, jnp.int32)]).reshape(-1, CH)
    dst = jnp.concatenate(
        [edge_index[1], jnp.full((EP - E,), N, jnp.int32)]).reshape(-1, CH)
    flat_idx = batch.astype(jnp.int32) * MAXN + k.astype(jnp.int32)
    fidx = jnp.concatenate([flat_idx, jnp.full((NP - N,), N, jnp.int32)])
    x_p = jnp.pad(x, ((0, NP - N), (0, 0)))
    wl_p = jnp.pad(W_leaf, ((0, 0), (0, LW - 2)))
    bl_p = jnp.pad(b_leaf, ((0, LW - 2),)).reshape(1, LW)
    zrow = jnp.zeros((ROWS_PER_TILE,), f32)
    ztile64 = jnp.zeros((CH, H), f32)
    ztile16 = jnp.zeros((CH, LW), f32)

    degp = _sc_deg(dst, zrow)
    deg2 = jnp.transpose(degp)
    h1, g1, norm = _tc1(deg2, x_p, W_bb)
    a1 = _sc_segsum64(g1, src, dst, ztile64)
    t2, g2 = _tc2(a1, h1, norm, b_bb.reshape(1, H), W_body)
    a2 = _sc_segsum64(g2, src, dst, ztile64)
    g3, sb3, ye = _tc3(a2, t2, norm, b_body.reshape(1, H), wl_p, bl_p,
                       W_eos, b_eos.reshape(1, 1))
    y16 = _sc_leaf(g3, src, dst, sb3, norm.reshape(NP), fidx, ztile16)
    y_leaf = y16[:N, :2].reshape(B, MAXN * 2)
    y_eos = ye.reshape(B)
    return (y_leaf, y_eos)


# speedup vs baseline: 14.7219x; 1.1025x over previous
import functools

import jax
import jax.numpy as jnp
from jax import lax
from jax.experimental import pallas as pl
from jax.experimental.pallas import tpu as pltpu
from jax.experimental.pallas import tpu_sc as plsc

N = 10000
E = 320000
D = 128
H = 64
B = 8
MAXN = 1250

NC, NS = 2, 16
NW = NC * NS
NP = 10240
ROWS_PER_TILE = NP // NS
CH = 128
EP = 327680
EDGES_PER_W = EP // NW
NCHUNK = EDGES_PER_W // CH

_mesh = plsc.VectorSubcoreMesh(core_axis_name="c", subcore_axis_name="s")
_sc_params = pltpu.CompilerParams(use_tc_tiling_on_sc=False,
                                  needs_layout_passes=False)


@functools.partial(
    pl.kernel,
    out_type=jax.ShapeDtypeStruct((NC, NP), jnp.float32),
    mesh=_mesh,
    compiler_params=_sc_params,
    scratch_types=[
        pltpu.VMEM_SHARED((NP,), jnp.float32),
        pltpu.VMEM((NCHUNK, CH), jnp.int32),
        pltpu.VMEM((CH,), jnp.float32),
        pltpu.VMEM((ROWS_PER_TILE,), jnp.float32),
        pltpu.SemaphoreType.DMA,
    ],
)
def _sc_deg(dst2_hbm, zrow_hbm, out_hbm, acc_sh, dstb_v, ones_v, zrow_v,
            dsem):
    c = lax.axis_index("c")
    s = lax.axis_index("s")
    wid = s * NC + c
    rbase = s * ROWS_PER_TILE
    pltpu.sync_copy(dst2_hbm.at[pl.ds(wid * NCHUNK, NCHUNK)], dstb_v)
    pltpu.sync_copy(zrow_hbm, zrow_v)
    pltpu.sync_copy(zrow_v, acc_sh.at[pl.ds(rbase, ROWS_PER_TILE)])
    for i in range(CH // 16):
        ones_v[pl.ds(i * 16, 16)] = jnp.ones((16,), jnp.float32)
    plsc.subcore_barrier()

    def chunk(i, carry):
        pltpu.async_copy(ones_v, acc_sh.at[dstb_v.at[i]], dsem, add=True)
        return carry

    lax.fori_loop(0, NCHUNK, chunk, 0)

    def drain(i, carry):
        pltpu.make_async_copy(ones_v, acc_sh.at[dstb_v.at[i]], dsem).wait()
        return carry

    lax.fori_loop(0, NCHUNK, drain, 0)
    plsc.subcore_barrier()
    pltpu.sync_copy(acc_sh.at[pl.ds(rbase, ROWS_PER_TILE)],
                    out_hbm.at[c, pl.ds(rbase, ROWS_PER_TILE)])


NBUF = 8


def _make_sc_segsum(width):
    @functools.partial(
        pl.kernel,
        out_type=jax.ShapeDtypeStruct((NC, NP, width), jnp.float32),
        mesh=_mesh,
        compiler_params=_sc_params,
        scratch_types=[
            pltpu.VMEM_SHARED((NP, width), jnp.float32),
            pltpu.VMEM((NCHUNK, CH), jnp.int32),
            pltpu.VMEM((NCHUNK, CH), jnp.int32),
            pltpu.VMEM((NBUF, CH, width), jnp.float32),
            pltpu.SemaphoreType.DMA((NBUF,)),
            pltpu.SemaphoreType.DMA((NBUF,)),
        ],
    )
    def _sc_segsum(g_hbm, src2_hbm, dst2_hbm, ztile_hbm, out_hbm,
                   acc_sh, srcb_v, dstb_v, bufs_v, gsem_a, ssem_a):
        gsems = [gsem_a.at[b] for b in range(NBUF)]
        ssems = [ssem_a.at[b] for b in range(NBUF)]
        c = lax.axis_index("c")
        s = lax.axis_index("s")
        wid = s * NC + c
        rbase = s * ROWS_PER_TILE
        pltpu.sync_copy(src2_hbm.at[pl.ds(wid * NCHUNK, NCHUNK)], srcb_v)
        pltpu.sync_copy(dst2_hbm.at[pl.ds(wid * NCHUNK, NCHUNK)], dstb_v)
        pltpu.sync_copy(ztile_hbm, bufs_v.at[0])
        for r in range(ROWS_PER_TILE // CH):
            pltpu.sync_copy(bufs_v.at[0], acc_sh.at[pl.ds(rbase + r * CH, CH)])
        plsc.subcore_barrier()

        for b in range(NBUF):
            pltpu.async_copy(g_hbm.at[srcb_v.at[b]], bufs_v.at[b], gsems[b])

        def sbody(q, carry):
            i0 = q * NBUF
            for b in range(NBUF):
                i = i0 + b
                pltpu.make_async_copy(g_hbm.at[srcb_v.at[i]],
                                      bufs_v.at[b], gsems[b]).wait()
                pltpu.async_copy(bufs_v.at[b], acc_sh.at[dstb_v.at[i]],
                                 ssems[b], add=True)
            for b in range(NBUF):
                i = i0 + b
                pltpu.make_async_copy(bufs_v.at[b],
                                      acc_sh.at[dstb_v.at[i]],
                                      ssems[b]).wait()
                nxt = i + NBUF

                @pl.when(nxt < NCHUNK)
                def _():
                    pltpu.async_copy(g_hbm.at[srcb_v.at[nxt]], bufs_v.at[b],
                                     gsems[b])
            return carry

        lax.fori_loop(0, NCHUNK // NBUF, sbody, 0)
        plsc.subcore_barrier()
        pltpu.sync_copy(acc_sh.at[pl.ds(rbase, ROWS_PER_TILE)],
                        out_hbm.at[c, pl.ds(rbase, ROWS_PER_TILE)])

    return _sc_segsum


_sc_segsum64 = _make_sc_segsum(H)


LW = 16
EDGES_PER_T3 = EP // NS
NCHUNK3 = EDGES_PER_T3 // CH


@functools.partial(
    pl.kernel,
    out_type=jax.ShapeDtypeStruct((NP, LW), jnp.float32),
    mesh=_mesh,
    compiler_params=_sc_params,
    scratch_types=[
        pltpu.VMEM_SHARED((NP, LW), jnp.float32),
        pltpu.VMEM((NCHUNK3, CH), jnp.int32),
        pltpu.VMEM((NCHUNK3, CH), jnp.int32),
        pltpu.VMEM((NBUF, CH, LW), jnp.float32),
        pltpu.VMEM((ROWS_PER_TILE, LW), jnp.float32),
        pltpu.VMEM((ROWS_PER_TILE, LW), jnp.float32),
        pltpu.VMEM((ROWS_PER_TILE,), jnp.float32),
        pltpu.VMEM((CH,), jnp.int32),
        pltpu.SemaphoreType.DMA((NBUF,)),
        pltpu.SemaphoreType.DMA((NBUF,)),
    ],
)
def _sc_leaf(g3_hbm, src2_hbm, dst2_hbm, sb3_hbm, norm_hbm, fidx_hbm,
             ztile_hbm, out_hbm, acc_sh, srcb_v, dstb_v, bufs_v, a3_v, sb_v,
             norm_v, fc_v, gsem_a, ssem_a):
    gsems = [gsem_a.at[b] for b in range(NBUF)]
    ssems = [ssem_a.at[b] for b in range(NBUF)]
    c = lax.axis_index("c")
    s = lax.axis_index("s")

    @pl.when(c == 1)
    def _():
        rbase = s * ROWS_PER_TILE
        pltpu.sync_copy(src2_hbm.at[pl.ds(s * NCHUNK3, NCHUNK3)], srcb_v)
        pltpu.sync_copy(dst2_hbm.at[pl.ds(s * NCHUNK3, NCHUNK3)], dstb_v)
        pltpu.sync_copy(ztile_hbm, bufs_v.at[0])
        for r in range(ROWS_PER_TILE // CH):
            pltpu.sync_copy(bufs_v.at[0], acc_sh.at[pl.ds(rbase + r * CH, CH)])
        plsc.subcore_barrier()

        for b in range(NBUF):
            pltpu.async_copy(g3_hbm.at[srcb_v.at[b]], bufs_v.at[b], gsems[b])

        def sbody(q, carry):
            i0 = q * NBUF
            for b in range(NBUF):
                i = i0 + b
                pltpu.make_async_copy(g3_hbm.at[srcb_v.at[i]],
                                      bufs_v.at[b], gsems[b]).wait()
                pltpu.async_copy(bufs_v.at[b], acc_sh.at[dstb_v.at[i]],
                                 ssems[b], add=True)
            for b in range(NBUF):
                i = i0 + b
                pltpu.make_async_copy(bufs_v.at[b],
                                      acc_sh.at[dstb_v.at[i]],
                                      ssems[b]).wait()
                nxt = i + NBUF

                @pl.when(nxt < NCHUNK3)
                def _():
                    pltpu.async_copy(g3_hbm.at[srcb_v.at[nxt]], bufs_v.at[b],
                                     gsems[b])
            return carry

        lax.fori_loop(0, NCHUNK3 // NBUF, sbody, 0)
        plsc.subcore_barrier()

        pltpu.sync_copy(acc_sh.at[pl.ds(rbase, ROWS_PER_TILE)], a3_v)
        pltpu.sync_copy(sb3_hbm.at[pl.ds(rbase, ROWS_PER_TILE)], sb_v)
        pltpu.sync_copy(norm_hbm.at[pl.ds(rbase, ROWS_PER_TILE)], norm_v)
        for gi in range(ROWS_PER_TILE // 16):
            rid = lax.iota(jnp.int32, 16) + gi * 16
            nv = norm_v[pl.ds(gi * 16, 16)]
            for j in range(2):
                jv = jnp.full((16,), j, jnp.int32)
                av = plsc.load_gather(a3_v, [rid, jv])
                sv = plsc.load_gather(sb_v, [rid, jv])
                plsc.store_scatter(a3_v, [rid, jv], nv * av + sv)
        for ci in range(ROWS_PER_TILE // CH):
            pltpu.sync_copy(fidx_hbm.at[pl.ds(rbase + ci * CH, CH)], fc_v)
            pltpu.async_copy(a3_v.at[pl.ds(ci * CH, CH)],
                             out_hbm.at[fc_v], gsems[0]).wait()


R = 1024


def _tc1_body(d_ref, x_ref, w_ref, h1_ref, g1_ref, n_ref):
    deg = d_ref[:, 0:1] + d_ref[:, 1:2] + 1.0
    norm = lax.rsqrt(deg)
    h1 = jnp.dot(x_ref[...], w_ref[...], preferred_element_type=jnp.float32)
    h1_ref[...] = h1
    g1_ref[...] = norm * h1
    n_ref[...] = norm


def _tc2_body(a_ref, h1_ref, n_ref, b_ref, w_ref, t2_ref, g2_ref):
    a = a_ref[0] + a_ref[1]
    n = n_ref[...]
    pre = n * a + (n * n) * h1_ref[...] + b_ref[...]
    t2 = jnp.dot(pre, w_ref[...], preferred_element_type=jnp.float32)
    t2_ref[...] = t2
    g2_ref[...] = n * t2


def _tc3_body(a_ref, t2_ref, n_ref, bb_ref, wl_ref, bl_ref, we_ref, be_ref,
              g3_ref, sb3_ref, ye_ref):
    i = pl.program_id(0)
    a = a_ref[0] + a_ref[1]
    n = n_ref[...]
    pre = n * a + (n * n) * t2_ref[...] + bb_ref[...]
    h2 = jnp.where(pre >= 0, pre, 0.01 * pre)
    t3 = jnp.dot(h2, wl_ref[...], preferred_element_type=jnp.float32)
    g3_ref[...] = n * t3
    sb3_ref[...] = (n * n) * t3 + bl_ref[...]
    v = jnp.dot(h2, we_ref[...], preferred_element_type=jnp.float32)
    rows = lax.broadcasted_iota(jnp.int32, (R, 1), 0) + i * R
    gid = rows // MAXN
    valid = rows < N

    @pl.when(i == 0)
    def _():
        ye_ref[...] = jnp.broadcast_to(be_ref[0, 0], (1, B))

    parts = [jnp.sum(jnp.where((gid == gg) & valid, v, 0.0))
             for gg in range(B)]
    ye_ref[...] = ye_ref[...] + jnp.stack(parts).reshape(1, B)


def _row_spec(width):
    return pl.BlockSpec((R, width), lambda i: (i, 0))


def _part_spec(width):
    return pl.BlockSpec((NC, R, width), lambda i: (0, i, 0))


def _full_spec(shape):
    return pl.BlockSpec(shape, lambda i: tuple(0 for _ in shape))


_tc1 = pl.pallas_call(
    _tc1_body,
    grid=(NP // R,),
    in_specs=[_row_spec(2), _row_spec(D), _full_spec((D, H))],
    out_specs=[_row_spec(H), _row_spec(H), _row_spec(1)],
    out_shape=[jax.ShapeDtypeStruct((NP, H), jnp.float32),
               jax.ShapeDtypeStruct((NP, H), jnp.float32),
               jax.ShapeDtypeStruct((NP, 1), jnp.float32)],
)

_tc2 = pl.pallas_call(
    _tc2_body,
    grid=(NP // R,),
    in_specs=[_part_spec(H), _row_spec(H), _row_spec(1),
              _full_spec((1, H)), _full_spec((H, H))],
    out_specs=[_row_spec(H), _row_spec(H)],
    out_shape=[jax.ShapeDtypeStruct((NP, H), jnp.float32),
               jax.ShapeDtypeStruct((NP, H), jnp.float32)],
)

_tc3 = pl.pallas_call(
    _tc3_body,
    grid=(NP // R,),
    in_specs=[_part_spec(H), _row_spec(H), _row_spec(1),
              _full_spec((1, H)), _full_spec((H, LW)), _full_spec((1, LW)),
              _full_spec((H, 1)), _full_spec((1, 1))],
    out_specs=[_row_spec(LW), _row_spec(LW),
               pl.BlockSpec((1, B), lambda i: (0, 0))],
    out_shape=[jax.ShapeDtypeStruct((NP, LW), jnp.float32),
               jax.ShapeDtypeStruct((NP, LW), jnp.float32),
               jax.ShapeDtypeStruct((1, B), jnp.float32)],
)


@jax.jit
def kernel(x, edge_index, k, batch, W_bb, b_bb, W_body, b_body,
           W_leaf, b_leaf, W_eos, b_eos):
    f32 = jnp.float32
    src = jnp.concatenate(
        [edge_index[0], jnp.full((EP - E,), N, jnp.int32)]).reshape(-1, CH)
    dst = jnp.concatenate(
        [edge_index[1], jnp.full((EP - E,), N, jnp.int32)]).reshape(-1, CH)
    flat_idx = batch.astype(jnp.int32) * MAXN + k.astype(jnp.int32)
    fidx = jnp.concatenate([flat_idx, jnp.full((NP - N,), N, jnp.int32)])
    x_p = jnp.pad(x, ((0, NP - N), (0, 0)))
    wl_p = jnp.pad(W_leaf, ((0, 0), (0, LW - 2)))
    bl_p = jnp.pad(b_leaf, ((0, LW - 2),)).reshape(1, LW)
    zrow = jnp.zeros((ROWS_PER_TILE,), f32)
    ztile64 = jnp.zeros((CH, H), f32)
    ztile16 = jnp.zeros((CH, LW), f32)

    degp = _sc_deg(dst, zrow)
    deg2 = jnp.transpose(degp)
    h1, g1, norm = _tc1(deg2, x_p, W_bb)
    a1 = _sc_segsum64(g1, src, dst, ztile64)
    t2, g2 = _tc2(a1, h1, norm, b_bb.reshape(1, H), W_body)
    a2 = _sc_segsum64(g2, src, dst, ztile64)
    g3, sb3, ye = _tc3(a2, t2, norm, b_body.reshape(1, H), wl_p, bl_p,
                       W_eos, b_eos.reshape(1, 1))
    y16 = _sc_leaf(g3, src, dst, sb3, norm.reshape(NP), fidx, ztile16)
    y_leaf = y16[:N, :2].reshape(B, MAXN * 2)
    y_eos = ye.reshape(B)
    return (y_leaf, y_eos)


# speedup vs baseline: 15.1439x; 1.0287x over previous
import functools

import jax
import jax.numpy as jnp
from jax import lax
from jax.experimental import pallas as pl
from jax.experimental.pallas import tpu as pltpu
from jax.experimental.pallas import tpu_sc as plsc

N = 10000
E = 320000
D = 128
H = 64
B = 8
MAXN = 1250

NC, NS = 2, 16
NW = NC * NS
NP = 10240
ROWS_PER_TILE = NP // NS
CH = 128
EP = 327680
EDGES_PER_W = EP // NW
NCHUNK = EDGES_PER_W // CH

_mesh = plsc.VectorSubcoreMesh(core_axis_name="c", subcore_axis_name="s")
_sc_params = pltpu.CompilerParams(use_tc_tiling_on_sc=False,
                                  needs_layout_passes=False)


@functools.partial(
    pl.kernel,
    out_type=jax.ShapeDtypeStruct((NC, NP), jnp.float32),
    mesh=_mesh,
    compiler_params=_sc_params,
    scratch_types=[
        pltpu.VMEM_SHARED((NP,), jnp.float32),
        pltpu.VMEM((NCHUNK, 2, CH), jnp.int32),
        pltpu.VMEM((CH,), jnp.float32),
        pltpu.VMEM((ROWS_PER_TILE,), jnp.float32),
        pltpu.SemaphoreType.DMA,
    ],
)
def _sc_deg(idx2_hbm, zrow_hbm, out_hbm, acc_sh, idxb_v, ones_v, zrow_v,
            dsem):
    c = lax.axis_index("c")
    s = lax.axis_index("s")
    wid = s * NC + c
    rbase = s * ROWS_PER_TILE
    pltpu.sync_copy(idx2_hbm.at[pl.ds(wid * NCHUNK, NCHUNK)], idxb_v)
    pltpu.sync_copy(zrow_hbm, zrow_v)
    pltpu.sync_copy(zrow_v, acc_sh.at[pl.ds(rbase, ROWS_PER_TILE)])
    for i in range(CH // 16):
        ones_v[pl.ds(i * 16, 16)] = jnp.ones((16,), jnp.float32)
    plsc.subcore_barrier()

    def chunk(i, carry):
        pltpu.async_copy(ones_v, acc_sh.at[idxb_v.at[i, 1]], dsem, add=True)
        return carry

    lax.fori_loop(0, NCHUNK, chunk, 0)

    def drain(i, carry):
        pltpu.make_async_copy(ones_v, acc_sh.at[idxb_v.at[i, 1]], dsem).wait()
        return carry

    lax.fori_loop(0, NCHUNK, drain, 0)
    plsc.subcore_barrier()
    pltpu.sync_copy(acc_sh.at[pl.ds(rbase, ROWS_PER_TILE)],
                    out_hbm.at[c, pl.ds(rbase, ROWS_PER_TILE)])


NBUF = 8
FAST_C = 1
FAST_BLKS = 128
SLOW_BLKS = 32
PAIR_BLKS = FAST_BLKS + SLOW_BLKS


def _seg_pipeline(g_hbm, idx2_hbm, acc_sh, idxb_v, bufs_v,
                  isems, gsems, ssems, blk0, nblk):
    for b in range(NBUF):
        pltpu.async_copy(idx2_hbm.at[blk0 + b], idxb_v.at[b], isems[b])
    for b in range(NBUF):
        pltpu.make_async_copy(idx2_hbm.at[blk0 + b], idxb_v.at[b],
                              isems[b]).wait()
        pltpu.async_copy(g_hbm.at[idxb_v.at[b, 0]], bufs_v.at[b], gsems[b])

    def sbody(q, carry):
        i0 = q * NBUF
        for b in range(NBUF):
            pltpu.make_async_copy(g_hbm.at[idxb_v.at[b, 0]],
                                  bufs_v.at[b], gsems[b]).wait()
            pltpu.async_copy(bufs_v.at[b], acc_sh.at[idxb_v.at[b, 1]],
                             ssems[b], add=True)
        for b in range(NBUF):
            nxt = i0 + b + NBUF
            pltpu.make_async_copy(bufs_v.at[b], acc_sh.at[idxb_v.at[b, 1]],
                                  ssems[b]).wait()

            @pl.when(nxt < nblk)
            def _():
                pltpu.async_copy(idx2_hbm.at[blk0 + nxt], idxb_v.at[b],
                                 isems[b])
        for b in range(NBUF):
            nxt = i0 + b + NBUF

            @pl.when(nxt < nblk)
            def _():
                pltpu.make_async_copy(idx2_hbm.at[blk0 + nxt], idxb_v.at[b],
                                      isems[b]).wait()
                pltpu.async_copy(g_hbm.at[idxb_v.at[b, 0]], bufs_v.at[b],
                                 gsems[b])
        return carry

    lax.fori_loop(0, nblk // NBUF, sbody, 0)


def _make_sc_segsum(width):
    @functools.partial(
        pl.kernel,
        out_type=jax.ShapeDtypeStruct((NC, NP, width), jnp.float32),
        mesh=_mesh,
        compiler_params=_sc_params,
        scratch_types=[
            pltpu.VMEM_SHARED((NP, width), jnp.float32),
            pltpu.VMEM((NBUF, 2, CH), jnp.int32),
            pltpu.VMEM((NBUF, CH, width), jnp.float32),
            pltpu.SemaphoreType.DMA((NBUF,)),
            pltpu.SemaphoreType.DMA((NBUF,)),
            pltpu.SemaphoreType.DMA((NBUF,)),
        ],
    )
    def _sc_segsum(g_hbm, idx2_hbm, ztile_hbm, out_hbm,
                   acc_sh, idxb_v, bufs_v, isem_a, gsem_a, ssem_a):
        isems = [isem_a.at[b] for b in range(NBUF)]
        gsems = [gsem_a.at[b] for b in range(NBUF)]
        ssems = [ssem_a.at[b] for b in range(NBUF)]
        c = lax.axis_index("c")
        s = lax.axis_index("s")
        rbase = s * ROWS_PER_TILE
        pltpu.sync_copy(ztile_hbm, bufs_v.at[0])
        for r in range(ROWS_PER_TILE // CH):
            pltpu.sync_copy(bufs_v.at[0], acc_sh.at[pl.ds(rbase + r * CH, CH)])
        plsc.subcore_barrier()

        @pl.when(c == FAST_C)
        def _():
            _seg_pipeline(g_hbm, idx2_hbm, acc_sh, idxb_v, bufs_v,
                          isems, gsems, ssems, s * PAIR_BLKS, FAST_BLKS)

        @pl.when(c != FAST_C)
        def _():
            _seg_pipeline(g_hbm, idx2_hbm, acc_sh, idxb_v, bufs_v,
                          isems, gsems, ssems, s * PAIR_BLKS + FAST_BLKS,
                          SLOW_BLKS)

        plsc.subcore_barrier()
        pltpu.sync_copy(acc_sh.at[pl.ds(rbase, ROWS_PER_TILE)],
                        out_hbm.at[c, pl.ds(rbase, ROWS_PER_TILE)])

    return _sc_segsum


_sc_segsum64 = _make_sc_segsum(H)


LW = 16
EDGES_PER_T3 = EP // NS
NCHUNK3 = EDGES_PER_T3 // CH


@functools.partial(
    pl.kernel,
    out_type=jax.ShapeDtypeStruct((NP, LW), jnp.float32),
    mesh=_mesh,
    compiler_params=_sc_params,
    scratch_types=[
        pltpu.VMEM_SHARED((NP, LW), jnp.float32),
        pltpu.VMEM((NBUF, 2, CH), jnp.int32),
        pltpu.VMEM((NBUF, CH, LW), jnp.float32),
        pltpu.VMEM((ROWS_PER_TILE, LW), jnp.float32),
        pltpu.VMEM((ROWS_PER_TILE, LW), jnp.float32),
        pltpu.VMEM((ROWS_PER_TILE,), jnp.float32),
        pltpu.VMEM((CH,), jnp.int32),
        pltpu.SemaphoreType.DMA((NBUF,)),
        pltpu.SemaphoreType.DMA((NBUF,)),
        pltpu.SemaphoreType.DMA((NBUF,)),
    ],
)
def _sc_leaf(g3_hbm, idx2_hbm, sb3_hbm, norm_hbm, fidx_hbm,
             ztile_hbm, out_hbm, acc_sh, idxb_v, bufs_v, a3_v, sb_v,
             norm_v, fc_v, isem_a, gsem_a, ssem_a):
    isems = [isem_a.at[b] for b in range(NBUF)]
    gsems = [gsem_a.at[b] for b in range(NBUF)]
    ssems = [ssem_a.at[b] for b in range(NBUF)]
    c = lax.axis_index("c")
    s = lax.axis_index("s")

    @pl.when(c == FAST_C)
    def _():
        rbase = s * ROWS_PER_TILE
        pltpu.sync_copy(ztile_hbm, bufs_v.at[0])
        for r in range(ROWS_PER_TILE // CH):
            pltpu.sync_copy(bufs_v.at[0], acc_sh.at[pl.ds(rbase + r * CH, CH)])
        plsc.subcore_barrier()
        _seg_pipeline(g3_hbm, idx2_hbm, acc_sh, idxb_v, bufs_v,
                      isems, gsems, ssems, s * NCHUNK3, NCHUNK3)
        plsc.subcore_barrier()

        pltpu.sync_copy(acc_sh.at[pl.ds(rbase, ROWS_PER_TILE)], a3_v)
        pltpu.sync_copy(sb3_hbm.at[pl.ds(rbase, ROWS_PER_TILE)], sb_v)
        pltpu.sync_copy(norm_hbm.at[pl.ds(rbase, ROWS_PER_TILE)], norm_v)
        for gi in range(ROWS_PER_TILE // 16):
            rid = lax.iota(jnp.int32, 16) + gi * 16
            nv = norm_v[pl.ds(gi * 16, 16)]
            for j in range(2):
                jv = jnp.full((16,), j, jnp.int32)
                av = plsc.load_gather(a3_v, [rid, jv])
                sv = plsc.load_gather(sb_v, [rid, jv])
                plsc.store_scatter(a3_v, [rid, jv], nv * av + sv)
        for ci in range(ROWS_PER_TILE // CH):
            pltpu.sync_copy(fidx_hbm.at[pl.ds(rbase + ci * CH, CH)], fc_v)
            pltpu.async_copy(a3_v.at[pl.ds(ci * CH, CH)],
                             out_hbm.at[fc_v], gsems[0]).wait()


R = 1024


def _tc1_body(d_ref, x_ref, w_ref, h1_ref, g1_ref, n_ref):
    deg = d_ref[:, 0:1] + d_ref[:, 1:2] + 1.0
    norm = lax.rsqrt(deg)
    h1 = jnp.dot(x_ref[...], w_ref[...], preferred_element_type=jnp.float32)
    h1_ref[...] = h1
    g1_ref[...] = norm * h1
    n_ref[...] = norm


def _tc2_body(a_ref, h1_ref, n_ref, b_ref, w_ref, t2_ref, g2_ref):
    a = a_ref[0] + a_ref[1]
    n = n_ref[...]
    pre = n * a + (n * n) * h1_ref[...] + b_ref[...]
    t2 = jnp.dot(pre, w_ref[...], preferred_element_type=jnp.float32)
    t2_ref[...] = t2
    g2_ref[...] = n * t2


def _tc3_body(a_ref, t2_ref, n_ref, bb_ref, wl_ref, bl_ref, we_ref, be_ref,
              g3_ref, sb3_ref, ye_ref):
    i = pl.program_id(0)
    a = a_ref[0] + a_ref[1]
    n = n_ref[...]
    pre = n * a + (n * n) * t2_ref[...] + bb_ref[...]
    h2 = jnp.where(pre >= 0, pre, 0.01 * pre)
    t3 = jnp.dot(h2, wl_ref[...], preferred_element_type=jnp.float32)
    g3_ref[...] = n * t3
    sb3_ref[...] = (n * n) * t3 + bl_ref[...]
    v = jnp.dot(h2, we_ref[...], preferred_element_type=jnp.float32)
    rows = lax.broadcasted_iota(jnp.int32, (R, 1), 0) + i * R
    gid = rows // MAXN
    valid = rows < N

    @pl.when(i == 0)
    def _():
        ye_ref[...] = jnp.broadcast_to(be_ref[0, 0], (1, B))

    parts = [jnp.sum(jnp.where((gid == gg) & valid, v, 0.0))
             for gg in range(B)]
    ye_ref[...] = ye_ref[...] + jnp.stack(parts).reshape(1, B)


def _row_spec(width):
    return pl.BlockSpec((R, width), lambda i: (i, 0))


def _part_spec(width):
    return pl.BlockSpec((NC, R, width), lambda i: (0, i, 0))


def _full_spec(shape):
    return pl.BlockSpec(shape, lambda i: tuple(0 for _ in shape))


_tc1 = pl.pallas_call(
    _tc1_body,
    grid=(NP // R,),
    in_specs=[_row_spec(2), _row_spec(D), _full_spec((D, H))],
    out_specs=[_row_spec(H), _row_spec(H), _row_spec(1)],
    out_shape=[jax.ShapeDtypeStruct((NP, H), jnp.float32),
               jax.ShapeDtypeStruct((NP, H), jnp.float32),
               jax.ShapeDtypeStruct((NP, 1), jnp.float32)],
)

_tc2 = pl.pallas_call(
    _tc2_body,
    grid=(NP // R,),
    in_specs=[_part_spec(H), _row_spec(H), _row_spec(1),
              _full_spec((1, H)), _full_spec((H, H))],
    out_specs=[_row_spec(H), _row_spec(H)],
    out_shape=[jax.ShapeDtypeStruct((NP, H), jnp.float32),
               jax.ShapeDtypeStruct((NP, H), jnp.float32)],
)

_tc3 = pl.pallas_call(
    _tc3_body,
    grid=(NP // R,),
    in_specs=[_part_spec(H), _row_spec(H), _row_spec(1),
              _full_spec((1, H)), _full_spec((H, LW)), _full_spec((1, LW)),
              _full_spec((H, 1)), _full_spec((1, 1))],
    out_specs=[_row_spec(LW), _row_spec(LW),
               pl.BlockSpec((1, B), lambda i: (0, 0))],
    out_shape=[jax.ShapeDtypeStruct((NP, LW), jnp.float32),
               jax.ShapeDtypeStruct((NP, LW), jnp.float32),
               jax.ShapeDtypeStruct((1, B), jnp.float32)],
)


@jax.jit
def kernel(x, edge_index, k, batch, W_bb, b_bb, W_body, b_body,
           W_leaf, b_leaf, W_eos, b_eos):
    f32 = jnp.float32
    ei_p = jnp.concatenate(
        [edge_index, jnp.full((2, EP - E), N, jnp.int32)], axis=1)
    idx2 = jnp.transpose(ei_p.reshape(2, EP // CH, CH), (1, 0, 2))
    flat_idx = batch.astype(jnp.int32) * MAXN + k.astype(jnp.int32)
    fidx = jnp.concatenate([flat_idx, jnp.full((NP - N,), N, jnp.int32)])
    x_p = jnp.pad(x, ((0, NP - N), (0, 0)))
    wl_p = jnp.pad(W_leaf, ((0, 0), (0, LW - 2)))
    bl_p = jnp.pad(b_leaf, ((0, LW - 2),)).reshape(1, LW)
    zrow = jnp.zeros((ROWS_PER_TILE,), f32)
    ztile64 = jnp.zeros((CH, H), f32)
    ztile16 = jnp.zeros((CH, LW), f32)

    degp = _sc_deg(idx2, zrow)
    deg2 = jnp.transpose(degp)
    h1, g1, norm = _tc1(deg2, x_p, W_bb)
    a1 = _sc_segsum64(g1, idx2, ztile64)
    t2, g2 = _tc2(a1, h1, norm, b_bb.reshape(1, H), W_body)
    a2 = _sc_segsum64(g2, idx2, ztile64)
    g3, sb3, ye = _tc3(a2, t2, norm, b_body.reshape(1, H), wl_p, bl_p,
                       W_eos, b_eos.reshape(1, 1))
    y16 = _sc_leaf(g3, idx2, sb3, norm.reshape(NP), fidx, ztile16)
    y_leaf = y16[:N, :2].reshape(B, MAXN * 2)
    y_eos = ye.reshape(B)
    return (y_leaf, y_eos)


# speedup vs baseline: 15.2240x; 1.0053x over previous
import functools

import jax
import jax.numpy as jnp
from jax import lax
from jax.experimental import pallas as pl
from jax.experimental.pallas import tpu as pltpu
from jax.experimental.pallas import tpu_sc as plsc

N = 10000
E = 320000
D = 128
H = 64
B = 8
MAXN = 1250

NC, NS = 2, 16
NW = NC * NS
NP = 10240
ROWS_PER_TILE = NP // NS
CH = 128
EP = 327680
EDGES_PER_W = EP // NW
NCHUNK = EDGES_PER_W // CH

_mesh = plsc.VectorSubcoreMesh(core_axis_name="c", subcore_axis_name="s")
_sc_params = pltpu.CompilerParams(use_tc_tiling_on_sc=False,
                                  needs_layout_passes=False)


@functools.partial(
    pl.kernel,
    out_type=jax.ShapeDtypeStruct((NC, NP), jnp.float32),
    mesh=_mesh,
    compiler_params=_sc_params,
    scratch_types=[
        pltpu.VMEM_SHARED((NP,), jnp.float32),
        pltpu.VMEM((NCHUNK, CH), jnp.int32),
        pltpu.VMEM((CH,), jnp.float32),
        pltpu.VMEM((ROWS_PER_TILE,), jnp.float32),
        pltpu.SemaphoreType.DMA,
    ],
)
def _sc_deg(dst2_hbm, zrow_hbm, out_hbm, acc_sh, dstb_v, ones_v, zrow_v,
            dsem):
    c = lax.axis_index("c")
    s = lax.axis_index("s")
    wid = s * NC + c
    rbase = s * ROWS_PER_TILE
    pltpu.sync_copy(dst2_hbm.at[pl.ds(wid * NCHUNK, NCHUNK)], dstb_v)
    pltpu.sync_copy(zrow_hbm, zrow_v)
    pltpu.sync_copy(zrow_v, acc_sh.at[pl.ds(rbase, ROWS_PER_TILE)])
    for i in range(CH // 16):
        ones_v[pl.ds(i * 16, 16)] = jnp.ones((16,), jnp.float32)
    plsc.subcore_barrier()

    def chunk(i, carry):
        pltpu.async_copy(ones_v, acc_sh.at[dstb_v.at[i]], dsem, add=True)
        return carry

    lax.fori_loop(0, NCHUNK, chunk, 0)

    def drain(i, carry):
        pltpu.make_async_copy(ones_v, acc_sh.at[dstb_v.at[i]], dsem).wait()
        return carry

    lax.fori_loop(0, NCHUNK, drain, 0)
    plsc.subcore_barrier()
    pltpu.sync_copy(acc_sh.at[pl.ds(rbase, ROWS_PER_TILE)],
                    out_hbm.at[c, pl.ds(rbase, ROWS_PER_TILE)])


NBUF = 5
LBUF = 8
FAST_C = 1
FAST_BLKS = 120
SLOW_BLKS = 40
PAIR_BLKS = FAST_BLKS + SLOW_BLKS


def _seg_pipeline(g_hbm, acc_sh, srcb_v, dstb_v, bufs_v, gsems, ssems,
                  nblk, nbuf):
    assert nblk % nbuf == 0, "chunks would leak undrained DMAs"
    for b in range(nbuf):
        pltpu.async_copy(g_hbm.at[srcb_v.at[b]], bufs_v.at[b], gsems[b])

    def sbody(q, carry):
        i0 = q * nbuf
        for b in range(nbuf):
            i = i0 + b
            pltpu.make_async_copy(g_hbm.at[srcb_v.at[i]],
                                  bufs_v.at[b], gsems[b]).wait()
            pltpu.async_copy(bufs_v.at[b], acc_sh.at[dstb_v.at[i]],
                             ssems[b], add=True)
        for b in range(nbuf):
            i = i0 + b
            pltpu.make_async_copy(bufs_v.at[b], acc_sh.at[dstb_v.at[i]],
                                  ssems[b]).wait()
            nxt = i + nbuf

            @pl.when(nxt < nblk)
            def _():
                pltpu.async_copy(g_hbm.at[srcb_v.at[nxt]], bufs_v.at[b],
                                 gsems[b])
        return carry

    lax.fori_loop(0, nblk // nbuf, sbody, 0)


def _make_sc_segsum(width):
    @functools.partial(
        pl.kernel,
        out_type=jax.ShapeDtypeStruct((NC, NP, width), jnp.float32),
        mesh=_mesh,
        compiler_params=_sc_params,
        scratch_types=[
            pltpu.VMEM_SHARED((NP, width), jnp.float32),
            pltpu.VMEM((FAST_BLKS, CH), jnp.int32),
            pltpu.VMEM((FAST_BLKS, CH), jnp.int32),
            pltpu.VMEM((NBUF, CH, width), jnp.float32),
            pltpu.SemaphoreType.DMA((NBUF,)),
            pltpu.SemaphoreType.DMA((NBUF,)),
        ],
    )
    def _sc_segsum(g_hbm, src2_hbm, dst2_hbm, ztile_hbm, out_hbm,
                   acc_sh, srcb_v, dstb_v, bufs_v, gsem_a, ssem_a):
        gsems = [gsem_a.at[b] for b in range(NBUF)]
        ssems = [ssem_a.at[b] for b in range(NBUF)]
        c = lax.axis_index("c")
        s = lax.axis_index("s")
        rbase = s * ROWS_PER_TILE
        pltpu.sync_copy(ztile_hbm, bufs_v.at[0])
        for r in range(ROWS_PER_TILE // CH):
            pltpu.sync_copy(bufs_v.at[0], acc_sh.at[pl.ds(rbase + r * CH, CH)])
        plsc.subcore_barrier()

        @pl.when(c == FAST_C)
        def _():
            blk0 = s * PAIR_BLKS
            pltpu.sync_copy(src2_hbm.at[pl.ds(blk0, FAST_BLKS)], srcb_v)
            pltpu.sync_copy(dst2_hbm.at[pl.ds(blk0, FAST_BLKS)], dstb_v)
            _seg_pipeline(g_hbm, acc_sh, srcb_v, dstb_v, bufs_v,
                          gsems, ssems, FAST_BLKS, NBUF)

        @pl.when(c != FAST_C)
        def _():
            blk0 = s * PAIR_BLKS + FAST_BLKS
            pltpu.sync_copy(src2_hbm.at[pl.ds(blk0, SLOW_BLKS)],
                            srcb_v.at[pl.ds(0, SLOW_BLKS)])
            pltpu.sync_copy(dst2_hbm.at[pl.ds(blk0, SLOW_BLKS)],
                            dstb_v.at[pl.ds(0, SLOW_BLKS)])
            _seg_pipeline(g_hbm, acc_sh, srcb_v, dstb_v, bufs_v,
                          gsems, ssems, SLOW_BLKS, NBUF)

        plsc.subcore_barrier()
        pltpu.sync_copy(acc_sh.at[pl.ds(rbase, ROWS_PER_TILE)],
                        out_hbm.at[c, pl.ds(rbase, ROWS_PER_TILE)])

    return _sc_segsum


_sc_segsum64 = _make_sc_segsum(H)


LW = 16
EDGES_PER_T3 = EP // NS
NCHUNK3 = EDGES_PER_T3 // CH


@functools.partial(
    pl.kernel,
    out_type=jax.ShapeDtypeStruct((NP, LW), jnp.float32),
    mesh=_mesh,
    compiler_params=_sc_params,
    scratch_types=[
        pltpu.VMEM_SHARED((NP, LW), jnp.float32),
        pltpu.VMEM((NCHUNK3, CH), jnp.int32),
        pltpu.VMEM((NCHUNK3, CH), jnp.int32),
        pltpu.VMEM((LBUF, CH, LW), jnp.float32),
        pltpu.VMEM((ROWS_PER_TILE, LW), jnp.float32),
        pltpu.VMEM((ROWS_PER_TILE, LW), jnp.float32),
        pltpu.VMEM((ROWS_PER_TILE,), jnp.float32),
        pltpu.VMEM((CH,), jnp.int32),
        pltpu.SemaphoreType.DMA((LBUF,)),
        pltpu.SemaphoreType.DMA((LBUF,)),
    ],
)
def _sc_leaf(g3_hbm, src2_hbm, dst2_hbm, sb3_hbm, norm_hbm, fidx_hbm,
             ztile_hbm, out_hbm, acc_sh, srcb_v, dstb_v, bufs_v, a3_v, sb_v,
             norm_v, fc_v, gsem_a, ssem_a):
    gsems = [gsem_a.at[b] for b in range(LBUF)]
    ssems = [ssem_a.at[b] for b in range(LBUF)]
    c = lax.axis_index("c")
    s = lax.axis_index("s")

    @pl.when(c == FAST_C)
    def _():
        rbase = s * ROWS_PER_TILE
        pltpu.sync_copy(ztile_hbm, bufs_v.at[0])
        for r in range(ROWS_PER_TILE // CH):
            pltpu.sync_copy(bufs_v.at[0], acc_sh.at[pl.ds(rbase + r * CH, CH)])
        pltpu.sync_copy(src2_hbm.at[pl.ds(s * NCHUNK3, NCHUNK3)], srcb_v)
        pltpu.sync_copy(dst2_hbm.at[pl.ds(s * NCHUNK3, NCHUNK3)], dstb_v)
        plsc.subcore_barrier()
        _seg_pipeline(g3_hbm, acc_sh, srcb_v, dstb_v, bufs_v,
                      gsems, ssems, NCHUNK3, LBUF)
        plsc.subcore_barrier()

        pltpu.sync_copy(acc_sh.at[pl.ds(rbase, ROWS_PER_TILE)], a3_v)
        pltpu.sync_copy(sb3_hbm.at[pl.ds(rbase, ROWS_PER_TILE)], sb_v)
        pltpu.sync_copy(norm_hbm.at[pl.ds(rbase, ROWS_PER_TILE)], norm_v)
        for gi in range(ROWS_PER_TILE // 16):
            rid = lax.iota(jnp.int32, 16) + gi * 16
            nv = norm_v[pl.ds(gi * 16, 16)]
            for j in range(2):
                jv = jnp.full((16,), j, jnp.int32)
                av = plsc.load_gather(a3_v, [rid, jv])
                sv = plsc.load_gather(sb_v, [rid, jv])
                plsc.store_scatter(a3_v, [rid, jv], nv * av + sv)
        for ci in range(ROWS_PER_TILE // CH):
            pltpu.sync_copy(fidx_hbm.at[pl.ds(rbase + ci * CH, CH)], fc_v)
            pltpu.async_copy(a3_v.at[pl.ds(ci * CH, CH)],
                             out_hbm.at[fc_v], gsems[0]).wait()


R = 1000


def _tc1_body(d_ref, x_ref, w_ref, h1_ref, g1_ref, n_ref):
    deg = d_ref[:, 0:1] + d_ref[:, 1:2] + 1.0
    norm = lax.rsqrt(deg)
    h1 = jnp.dot(x_ref[...], w_ref[...], preferred_element_type=jnp.float32)
    h1_ref[...] = h1
    g1_ref[...] = norm * h1
    n_ref[...] = norm


def _tc2_body(a_ref, h1_ref, n_ref, b_ref, w_ref, t2_ref, g2_ref):
    a = a_ref[0] + a_ref[1]
    n = n_ref[...]
    pre = n * a + (n * n) * h1_ref[...] + b_ref[...]
    t2 = jnp.dot(pre, w_ref[...], preferred_element_type=jnp.float32)
    t2_ref[...] = t2
    g2_ref[...] = n * t2


def _tc3_body(a_ref, t2_ref, n_ref, bb_ref, wl_ref, bl_ref, we_ref, be_ref,
              g3_ref, sb3_ref, ye_ref):
    i = pl.program_id(0)
    a = a_ref[0] + a_ref[1]
    n = n_ref[...]
    pre = n * a + (n * n) * t2_ref[...] + bb_ref[...]
    h2 = jnp.where(pre >= 0, pre, 0.01 * pre)
    t3 = jnp.dot(h2, wl_ref[...], preferred_element_type=jnp.float32)
    g3_ref[...] = n * t3
    sb3_ref[...] = (n * n) * t3 + bl_ref[...]
    v = jnp.dot(h2, we_ref[...], preferred_element_type=jnp.float32)
    rows = lax.broadcasted_iota(jnp.int32, (R, 1), 0) + i * R
    gid = rows // MAXN
    valid = rows < N

    @pl.when(i == 0)
    def _():
        ye_ref[...] = jnp.broadcast_to(be_ref[0, 0], (1, B))

    gcols = lax.broadcasted_iota(jnp.int32, (R, B), 1)
    onehot = jnp.where((gid == gcols) & valid, 1.0, 0.0)
    ye_ref[...] = ye_ref[...] + jnp.sum(v * onehot, axis=0, keepdims=True)


def _row_spec(width):
    return pl.BlockSpec((R, width), lambda i: (i, 0))


def _part_spec(width):
    return pl.BlockSpec((NC, R, width), lambda i: (0, i, 0))


def _full_spec(shape):
    return pl.BlockSpec(shape, lambda i: tuple(0 for _ in shape))


_tc1 = pl.pallas_call(
    _tc1_body,
    grid=(NP // R,),
    in_specs=[_row_spec(2), _row_spec(D), _full_spec((D, H))],
    out_specs=[_row_spec(H), _row_spec(H), _row_spec(1)],
    out_shape=[jax.ShapeDtypeStruct((NP, H), jnp.float32),
               jax.ShapeDtypeStruct((NP, H), jnp.float32),
               jax.ShapeDtypeStruct((NP, 1), jnp.float32)],
)

_tc2 = pl.pallas_call(
    _tc2_body,
    grid=(NP // R,),
    in_specs=[_part_spec(H), _row_spec(H), _row_spec(1),
              _full_spec((1, H)), _full_spec((H, H))],
    out_specs=[_row_spec(H), _row_spec(H)],
    out_shape=[jax.ShapeDtypeStruct((NP, H), jnp.float32),
               jax.ShapeDtypeStruct((NP, H), jnp.float32)],
)

_tc3 = pl.pallas_call(
    _tc3_body,
    grid=(NP // R,),
    in_specs=[_part_spec(H), _row_spec(H), _row_spec(1),
              _full_spec((1, H)), _full_spec((H, LW)), _full_spec((1, LW)),
              _full_spec((H, 1)), _full_spec((1, 1))],
    out_specs=[_row_spec(LW), _row_spec(LW),
               pl.BlockSpec((1, B), lambda i: (0, 0))],
    out_shape=[jax.ShapeDtypeStruct((NP, LW), jnp.float32),
               jax.ShapeDtypeStruct((NP, LW), jnp.float32),
               jax.ShapeDtypeStruct((1, B), jnp.float32)],
)


@jax.jit
def kernel(x, edge_index, k, batch, W_bb, b_bb, W_body, b_body,
           W_leaf, b_leaf, W_eos, b_eos):
    f32 = jnp.float32
    src2 = jnp.concatenate(
        [edge_index[0], jnp.full((EP - E,), N, jnp.int32)]).reshape(-1, CH)
    dst2 = jnp.concatenate(
        [edge_index[1], jnp.full((EP - E,), N, jnp.int32)]).reshape(-1, CH)
    flat_idx = batch.astype(jnp.int32) * MAXN + k.astype(jnp.int32)
    fidx = jnp.concatenate([flat_idx, jnp.full((NP - N,), N, jnp.int32)])
    wl_p = jnp.pad(W_leaf, ((0, 0), (0, LW - 2)))
    bl_p = jnp.pad(b_leaf, ((0, LW - 2),)).reshape(1, LW)
    zrow = jnp.zeros((ROWS_PER_TILE,), f32)
    ztile64 = jnp.zeros((CH, H), f32)
    ztile16 = jnp.zeros((CH, LW), f32)

    degp = _sc_deg(dst2, zrow)
    deg2 = jnp.transpose(degp)
    h1, g1, norm = _tc1(deg2, x, W_bb)
    a1 = _sc_segsum64(g1, src2, dst2, ztile64)
    t2, g2 = _tc2(a1, h1, norm, b_bb.reshape(1, H), W_body)
    a2 = _sc_segsum64(g2, src2, dst2, ztile64)
    g3, sb3, ye = _tc3(a2, t2, norm, b_body.reshape(1, H), wl_p, bl_p,
                       W_eos, b_eos.reshape(1, 1))
    y16 = _sc_leaf(g3, src2, dst2, sb3, norm.reshape(NP), fidx, ztile16)
    y_leaf = y16[:N, :2].reshape(B, MAXN * 2)
    y_eos = ye.reshape(B)
    return (y_leaf, y_eos)


# speedup vs baseline: 15.2974x; 1.0048x over previous
import functools

import jax
import jax.numpy as jnp
from jax import lax
from jax.experimental import pallas as pl
from jax.experimental.pallas import tpu as pltpu
from jax.experimental.pallas import tpu_sc as plsc

N = 10000
E = 320000
D = 128
H = 64
B = 8
MAXN = 1250

NC, NS = 2, 16
NW = NC * NS
NP = 10240
ROWS_PER_TILE = NP // NS
CH = 128
EP = 327680
EDGES_PER_W = EP // NW
NCHUNK = EDGES_PER_W // CH

_mesh = plsc.VectorSubcoreMesh(core_axis_name="c", subcore_axis_name="s")
_sc_params = pltpu.CompilerParams(use_tc_tiling_on_sc=False,
                                  needs_layout_passes=False)


@functools.partial(
    pl.kernel,
    out_type=jax.ShapeDtypeStruct((NC, NP), jnp.float32),
    mesh=_mesh,
    compiler_params=_sc_params,
    scratch_types=[
        pltpu.VMEM_SHARED((NP,), jnp.float32),
        pltpu.VMEM((NCHUNK, CH), jnp.int32),
        pltpu.VMEM((CH,), jnp.float32),
        pltpu.VMEM((ROWS_PER_TILE,), jnp.float32),
        pltpu.SemaphoreType.DMA,
    ],
)
def _sc_deg(dst2_hbm, zrow_hbm, out_hbm, acc_sh, dstb_v, ones_v, zrow_v,
            dsem):
    c = lax.axis_index("c")
    s = lax.axis_index("s")
    wid = s * NC + c
    rbase = s * ROWS_PER_TILE
    pltpu.sync_copy(dst2_hbm.at[pl.ds(wid * NCHUNK, NCHUNK)], dstb_v)
    pltpu.sync_copy(zrow_hbm, zrow_v)
    pltpu.sync_copy(zrow_v, acc_sh.at[pl.ds(rbase, ROWS_PER_TILE)])
    for i in range(CH // 16):
        ones_v[pl.ds(i * 16, 16)] = jnp.ones((16,), jnp.float32)
    plsc.subcore_barrier()

    def chunk(i, carry):
        pltpu.async_copy(ones_v, acc_sh.at[dstb_v.at[i]], dsem, add=True)
        return carry

    lax.fori_loop(0, NCHUNK, chunk, 0)

    def drain(i, carry):
        pltpu.make_async_copy(ones_v, acc_sh.at[dstb_v.at[i]], dsem).wait()
        return carry

    lax.fori_loop(0, NCHUNK, drain, 0)
    plsc.subcore_barrier()
    pltpu.sync_copy(acc_sh.at[pl.ds(rbase, ROWS_PER_TILE)],
                    out_hbm.at[c, pl.ds(rbase, ROWS_PER_TILE)])


NBUF = 8
LBUF = 8
FAST_C = 1
FAST_BLKS = 120
SLOW_BLKS = 40
PAIR_BLKS = FAST_BLKS + SLOW_BLKS


def _unpack_idx(pckb_v, uidx_v, i, b):
    for k in range(CH // 16):
        w = pckb_v[i, pl.ds(k * 16, 16)]
        uidx_v[b, 0, pl.ds(k * 16, 16)] = lax.bitwise_and(
            w, jnp.full((16,), 0xFFFF, jnp.int32))
        uidx_v[b, 1, pl.ds(k * 16, 16)] = lax.shift_right_logical(
            w, jnp.full((16,), 16, jnp.int32))


def _seg_pipeline(g_hbm, acc_sh, pckb_v, uidx_v, bufs_v, gsems, ssems,
                  nblk, nbuf):
    assert nblk % nbuf == 0, "chunks would leak undrained DMAs"
    for b in range(nbuf):
        _unpack_idx(pckb_v, uidx_v, b, b)
        pltpu.async_copy(g_hbm.at[uidx_v.at[b, 0]], bufs_v.at[b], gsems[b])

    def sbody(q, carry):
        i0 = q * nbuf
        for b in range(nbuf):
            pltpu.make_async_copy(g_hbm.at[uidx_v.at[b, 0]],
                                  bufs_v.at[b], gsems[b]).wait()
            pltpu.async_copy(bufs_v.at[b], acc_sh.at[uidx_v.at[b, 1]],
                             ssems[b], add=True)
        for b in range(nbuf):
            nxt = i0 + b + nbuf
            pltpu.make_async_copy(bufs_v.at[b], acc_sh.at[uidx_v.at[b, 1]],
                                  ssems[b]).wait()

            @pl.when(nxt < nblk)
            def _():
                _unpack_idx(pckb_v, uidx_v, nxt, b)
                pltpu.async_copy(g_hbm.at[uidx_v.at[b, 0]], bufs_v.at[b],
                                 gsems[b])
        return carry

    lax.fori_loop(0, nblk // nbuf, sbody, 0)


def _make_sc_segsum(width):
    @functools.partial(
        pl.kernel,
        out_type=jax.ShapeDtypeStruct((NC, NP, width), jnp.float32),
        mesh=_mesh,
        compiler_params=_sc_params,
        scratch_types=[
            pltpu.VMEM_SHARED((NP, width), jnp.float32),
            pltpu.VMEM((FAST_BLKS, CH), jnp.int32),
            pltpu.VMEM((NBUF, 2, CH), jnp.int32),
            pltpu.VMEM((NBUF, CH, width), jnp.float32),
            pltpu.SemaphoreType.DMA((NBUF,)),
            pltpu.SemaphoreType.DMA((NBUF,)),
        ],
    )
    def _sc_segsum(g_hbm, pck_hbm, ztile_hbm, out_hbm,
                   acc_sh, pckb_v, uidx_v, bufs_v, gsem_a, ssem_a):
        gsems = [gsem_a.at[b] for b in range(NBUF)]
        ssems = [ssem_a.at[b] for b in range(NBUF)]
        c = lax.axis_index("c")
        s = lax.axis_index("s")
        rbase = s * ROWS_PER_TILE
        pltpu.sync_copy(ztile_hbm, bufs_v.at[0])
        for r in range(ROWS_PER_TILE // CH):
            pltpu.sync_copy(bufs_v.at[0], acc_sh.at[pl.ds(rbase + r * CH, CH)])
        plsc.subcore_barrier()

        @pl.when(c == FAST_C)
        def _():
            blk0 = s * PAIR_BLKS
            pltpu.sync_copy(pck_hbm.at[pl.ds(blk0, FAST_BLKS)], pckb_v)
            _seg_pipeline(g_hbm, acc_sh, pckb_v, uidx_v, bufs_v,
                          gsems, ssems, FAST_BLKS, NBUF)

        @pl.when(c != FAST_C)
        def _():
            blk0 = s * PAIR_BLKS + FAST_BLKS
            pltpu.sync_copy(pck_hbm.at[pl.ds(blk0, SLOW_BLKS)],
                            pckb_v.at[pl.ds(0, SLOW_BLKS)])
            _seg_pipeline(g_hbm, acc_sh, pckb_v, uidx_v, bufs_v,
                          gsems, ssems, SLOW_BLKS, NBUF)

        plsc.subcore_barrier()
        pltpu.sync_copy(acc_sh.at[pl.ds(rbase, ROWS_PER_TILE)],
                        out_hbm.at[c, pl.ds(rbase, ROWS_PER_TILE)])

    return _sc_segsum


_sc_segsum64 = _make_sc_segsum(H)


LW = 16
EDGES_PER_T3 = EP // NS
NCHUNK3 = EDGES_PER_T3 // CH


@functools.partial(
    pl.kernel,
    out_type=jax.ShapeDtypeStruct((NP, LW), jnp.float32),
    mesh=_mesh,
    compiler_params=_sc_params,
    scratch_types=[
        pltpu.VMEM_SHARED((NP, LW), jnp.float32),
        pltpu.VMEM((NCHUNK3, CH), jnp.int32),
        pltpu.VMEM((LBUF, 2, CH), jnp.int32),
        pltpu.VMEM((LBUF, CH, LW), jnp.float32),
        pltpu.VMEM((ROWS_PER_TILE, LW), jnp.float32),
        pltpu.VMEM((ROWS_PER_TILE, LW), jnp.float32),
        pltpu.VMEM((ROWS_PER_TILE,), jnp.float32),
        pltpu.VMEM((CH,), jnp.int32),
        pltpu.SemaphoreType.DMA((LBUF,)),
        pltpu.SemaphoreType.DMA((LBUF,)),
    ],
)
def _sc_leaf(g3_hbm, pck_hbm, sb3_hbm, norm_hbm, fidx_hbm,
             ztile_hbm, out_hbm, acc_sh, pckb_v, uidx_v, bufs_v, a3_v, sb_v,
             norm_v, fc_v, gsem_a, ssem_a):
    gsems = [gsem_a.at[b] for b in range(LBUF)]
    ssems = [ssem_a.at[b] for b in range(LBUF)]
    c = lax.axis_index("c")
    s = lax.axis_index("s")

    @pl.when(c == FAST_C)
    def _():
        rbase = s * ROWS_PER_TILE
        pltpu.sync_copy(ztile_hbm, bufs_v.at[0])
        for r in range(ROWS_PER_TILE // CH):
            pltpu.sync_copy(bufs_v.at[0], acc_sh.at[pl.ds(rbase + r * CH, CH)])
        pltpu.sync_copy(pck_hbm.at[pl.ds(s * NCHUNK3, NCHUNK3)], pckb_v)
        plsc.subcore_barrier()
        _seg_pipeline(g3_hbm, acc_sh, pckb_v, uidx_v, bufs_v,
                      gsems, ssems, NCHUNK3, LBUF)
        plsc.subcore_barrier()

        pltpu.sync_copy(acc_sh.at[pl.ds(rbase, ROWS_PER_TILE)], a3_v)
        pltpu.sync_copy(sb3_hbm.at[pl.ds(rbase, ROWS_PER_TILE)], sb_v)
        pltpu.sync_copy(norm_hbm.at[pl.ds(rbase, ROWS_PER_TILE)], norm_v)
        for gi in range(ROWS_PER_TILE // 16):
            rid = lax.iota(jnp.int32, 16) + gi * 16
            nv = norm_v[pl.ds(gi * 16, 16)]
            for j in range(2):
                jv = jnp.full((16,), j, jnp.int32)
                av = plsc.load_gather(a3_v, [rid, jv])
                sv = plsc.load_gather(sb_v, [rid, jv])
                plsc.store_scatter(a3_v, [rid, jv], nv * av + sv)
        for ci in range(ROWS_PER_TILE // CH):
            pltpu.sync_copy(fidx_hbm.at[pl.ds(rbase + ci * CH, CH)], fc_v)
            pltpu.async_copy(a3_v.at[pl.ds(ci * CH, CH)],
                             out_hbm.at[fc_v], gsems[0]).wait()


R = 1000


def _tc1_body(d_ref, x_ref, w_ref, h1_ref, g1_ref, n_ref):
    deg = d_ref[:, 0:1] + d_ref[:, 1:2] + 1.0
    norm = lax.rsqrt(deg)
    h1 = jnp.dot(x_ref[...], w_ref[...], preferred_element_type=jnp.float32)
    h1_ref[...] = h1
    g1_ref[...] = norm * h1
    n_ref[...] = norm


def _tc2_body(a_ref, h1_ref, n_ref, b_ref, w_ref, t2_ref, g2_ref):
    a = a_ref[0] + a_ref[1]
    n = n_ref[...]
    pre = n * a + (n * n) * h1_ref[...] + b_ref[...]
    t2 = jnp.dot(pre, w_ref[...], preferred_element_type=jnp.float32)
    t2_ref[...] = t2
    g2_ref[...] = n * t2


def _tc3_body(a_ref, t2_ref, n_ref, bb_ref, wl_ref, bl_ref, we_ref, be_ref,
              g3_ref, sb3_ref, ye_ref):
    i = pl.program_id(0)
    a = a_ref[0] + a_ref[1]
    n = n_ref[...]
    pre = n * a + (n * n) * t2_ref[...] + bb_ref[...]
    h2 = jnp.where(pre >= 0, pre, 0.01 * pre)
    t3 = jnp.dot(h2, wl_ref[...], preferred_element_type=jnp.float32)
    g3_ref[...] = n * t3
    sb3_ref[...] = (n * n) * t3 + bl_ref[...]
    v = jnp.dot(h2, we_ref[...], preferred_element_type=jnp.float32)
    rows = lax.broadcasted_iota(jnp.int32, (R, 1), 0) + i * R
    gid = rows // MAXN
    valid = rows < N

    @pl.when(i == 0)
    def _():
        ye_ref[...] = jnp.broadcast_to(be_ref[0, 0], (1, B))

    gcols = lax.broadcasted_iota(jnp.int32, (R, B), 1)
    onehot = jnp.where((gid == gcols) & valid, 1.0, 0.0)
    ye_ref[...] = ye_ref[...] + jnp.sum(v * onehot, axis=0, keepdims=True)


def _row_spec(width):
    return pl.BlockSpec((R, width), lambda i: (i, 0))


def _part_spec(width):
    return pl.BlockSpec((NC, R, width), lambda i: (0, i, 0))


def _full_spec(shape):
    return pl.BlockSpec(shape, lambda i: tuple(0 for _ in shape))


_tc1 = pl.pallas_call(
    _tc1_body,
    grid=(NP // R,),
    in_specs=[_row_spec(2), _row_spec(D), _full_spec((D, H))],
    out_specs=[_row_spec(H), _row_spec(H), _row_spec(1)],
    out_shape=[jax.ShapeDtypeStruct((NP, H), jnp.float32),
               jax.ShapeDtypeStruct((NP, H), jnp.float32),
               jax.ShapeDtypeStruct((NP, 1), jnp.float32)],
)

_tc2 = pl.pallas_call(
    _tc2_body,
    grid=(NP // R,),
    in_specs=[_part_spec(H), _row_spec(H), _row_spec(1),
              _full_spec((1, H)), _full_spec((H, H))],
    out_specs=[_row_spec(H), _row_spec(H)],
    out_shape=[jax.ShapeDtypeStruct((NP, H), jnp.float32),
               jax.ShapeDtypeStruct((NP, H), jnp.float32)],
)

_tc3 = pl.pallas_call(
    _tc3_body,
    grid=(NP // R,),
    in_specs=[_part_spec(H), _row_spec(H), _row_spec(1),
              _full_spec((1, H)), _full_spec((H, LW)), _full_spec((1, LW)),
              _full_spec((H, 1)), _full_spec((1, 1))],
    out_specs=[_row_spec(LW), _row_spec(LW),
               pl.BlockSpec((1, B), lambda i: (0, 0))],
    out_shape=[jax.ShapeDtypeStruct((NP, LW), jnp.float32),
               jax.ShapeDtypeStruct((NP, LW), jnp.float32),
               jax.ShapeDtypeStruct((1, B), jnp.float32)],
)


@jax.jit
def kernel(x, edge_index, k, batch, W_bb, b_bb, W_body, b_body,
           W_leaf, b_leaf, W_eos, b_eos):
    f32 = jnp.float32
    src2 = jnp.concatenate(
        [edge_index[0], jnp.full((EP - E,), N, jnp.int32)]).reshape(-1, CH)
    dst2 = jnp.concatenate(
        [edge_index[1], jnp.full((EP - E,), N, jnp.int32)]).reshape(-1, CH)
    pck = (dst2 << 16) | src2
    flat_idx = batch.astype(jnp.int32) * MAXN + k.astype(jnp.int32)
    fidx = jnp.concatenate([flat_idx, jnp.full((NP - N,), N, jnp.int32)])
    wl_p = jnp.pad(W_leaf, ((0, 0), (0, LW - 2)))
    bl_p = jnp.pad(b_leaf, ((0, LW - 2),)).reshape(1, LW)
    zrow = jnp.zeros((ROWS_PER_TILE,), f32)
    ztile64 = jnp.zeros((CH, H), f32)
    ztile16 = jnp.zeros((CH, LW), f32)

    degp = _sc_deg(dst2, zrow)
    deg2 = jnp.transpose(degp)
    h1, g1, norm = _tc1(deg2, x, W_bb)
    a1 = _sc_segsum64(g1, pck, ztile64)
    t2, g2 = _tc2(a1, h1, norm, b_bb.reshape(1, H), W_body)
    a2 = _sc_segsum64(g2, pck, ztile64)
    g3, sb3, ye = _tc3(a2, t2, norm, b_body.reshape(1, H), wl_p, bl_p,
                       W_eos, b_eos.reshape(1, 1))
    y16 = _sc_leaf(g3, pck, sb3, norm.reshape(NP), fidx, ztile16)
    y_leaf = y16[:N, :2].reshape(B, MAXN * 2)
    y_eos = ye.reshape(B)
    return (y_leaf, y_eos)


# speedup vs baseline: 15.5754x; 1.0182x over previous
import functools

import jax
import jax.numpy as jnp
from jax import lax
from jax.experimental import pallas as pl
from jax.experimental.pallas import tpu as pltpu
from jax.experimental.pallas import tpu_sc as plsc

N = 10000
E = 320000
D = 128
H = 64
B = 8
MAXN = 1250

NC, NS = 2, 16
NW = NC * NS
NP = 10240
ROWS_PER_TILE = NP // NS
CH = 128
EP = 327680
EDGES_PER_W = EP // NW
NCHUNK = EDGES_PER_W // CH

_mesh = plsc.VectorSubcoreMesh(core_axis_name="c", subcore_axis_name="s")
_sc_params = pltpu.CompilerParams(use_tc_tiling_on_sc=False,
                                  needs_layout_passes=False)


@functools.partial(
    pl.kernel,
    out_type=jax.ShapeDtypeStruct((NC, NP), jnp.float32),
    mesh=_mesh,
    compiler_params=_sc_params,
    scratch_types=[
        pltpu.VMEM_SHARED((NP,), jnp.float32),
        pltpu.VMEM((NCHUNK, CH), jnp.int32),
        pltpu.VMEM((CH,), jnp.float32),
        pltpu.VMEM((ROWS_PER_TILE,), jnp.float32),
        pltpu.SemaphoreType.DMA,
    ],
)
def _sc_deg(dst2_hbm, zrow_hbm, out_hbm, acc_sh, dstb_v, ones_v, zrow_v,
            dsem):
    c = lax.axis_index("c")
    s = lax.axis_index("s")
    wid = s * NC + c
    rbase = s * ROWS_PER_TILE
    pltpu.sync_copy(dst2_hbm.at[pl.ds(wid * NCHUNK, NCHUNK)], dstb_v)
    pltpu.sync_copy(zrow_hbm, zrow_v)
    pltpu.sync_copy(zrow_v, acc_sh.at[pl.ds(rbase, ROWS_PER_TILE)])
    for i in range(CH // 16):
        ones_v[pl.ds(i * 16, 16)] = jnp.ones((16,), jnp.float32)
    plsc.subcore_barrier()

    def chunk(i, carry):
        pltpu.async_copy(ones_v, acc_sh.at[dstb_v.at[i]], dsem, add=True)
        return carry

    lax.fori_loop(0, NCHUNK, chunk, 0)

    def drain(i, carry):
        pltpu.make_async_copy(ones_v, acc_sh.at[dstb_v.at[i]], dsem).wait()
        return carry

    lax.fori_loop(0, NCHUNK, drain, 0)
    plsc.subcore_barrier()
    pltpu.sync_copy(acc_sh.at[pl.ds(rbase, ROWS_PER_TILE)],
                    out_hbm.at[c, pl.ds(rbase, ROWS_PER_TILE)])


NBUF = 8
LBUF = 8
FAST_C = 1
FAST_BLKS = 120
SLOW_BLKS = 40
PAIR_BLKS = FAST_BLKS + SLOW_BLKS


def _unpack_idx(pckb_v, uidx_v, i, b):
    for k in range(CH // 16):
        w = pckb_v[i, pl.ds(k * 16, 16)]
        uidx_v[b, 0, pl.ds(k * 16, 16)] = lax.bitwise_and(
            w, jnp.full((16,), 0xFFFF, jnp.int32))
        uidx_v[b, 1, pl.ds(k * 16, 16)] = lax.shift_right_logical(
            w, jnp.full((16,), 16, jnp.int32))


def _seg_pipeline(g_hbm, acc_sh, pckb_v, uidx_v, bufs_v, gsems, ssems,
                  nblk, nbuf):
    assert nblk % nbuf == 0, "chunks would leak undrained DMAs"
    for b in range(nbuf):
        _unpack_idx(pckb_v, uidx_v, b, b)
        pltpu.async_copy(g_hbm.at[uidx_v.at[b, 0]], bufs_v.at[b], gsems[b])

    def sbody(q, carry):
        i0 = q * nbuf
        for b in range(nbuf):
            pltpu.make_async_copy(g_hbm.at[uidx_v.at[b, 0]],
                                  bufs_v.at[b], gsems[b]).wait()
            pltpu.async_copy(bufs_v.at[b], acc_sh.at[uidx_v.at[b, 1]],
                             ssems[b], add=True)
        for b in range(nbuf):
            nxt = i0 + b + nbuf
            pltpu.make_async_copy(bufs_v.at[b], acc_sh.at[uidx_v.at[b, 1]],
                                  ssems[b]).wait()

            @pl.when(nxt < nblk)
            def _():
                _unpack_idx(pckb_v, uidx_v, nxt, b)
                pltpu.async_copy(g_hbm.at[uidx_v.at[b, 0]], bufs_v.at[b],
                                 gsems[b])
        return carry

    lax.fori_loop(0, nblk // nbuf, sbody, 0)


def _make_sc_segsum(width):
    @functools.partial(
        pl.kernel,
        out_type=jax.ShapeDtypeStruct((NC, NP, width), jnp.float32),
        mesh=_mesh,
        compiler_params=_sc_params,
        scratch_types=[
            pltpu.VMEM_SHARED((NP, width), jnp.float32),
            pltpu.VMEM((FAST_BLKS, CH), jnp.int32),
            pltpu.VMEM((NBUF, 2, CH), jnp.int32),
            pltpu.VMEM((NBUF, CH, width), jnp.float32),
            pltpu.SemaphoreType.DMA((NBUF,)),
            pltpu.SemaphoreType.DMA((NBUF,)),
        ],
    )
    def _sc_segsum(g_hbm, pck_hbm, ztile_hbm, out_hbm,
                   acc_sh, pckb_v, uidx_v, bufs_v, gsem_a, ssem_a):
        gsems = [gsem_a.at[b] for b in range(NBUF)]
        ssems = [ssem_a.at[b] for b in range(NBUF)]
        c = lax.axis_index("c")
        s = lax.axis_index("s")
        rbase = s * ROWS_PER_TILE
        pltpu.sync_copy(ztile_hbm, bufs_v.at[0])
        for r in range(ROWS_PER_TILE // CH):
            pltpu.sync_copy(bufs_v.at[0], acc_sh.at[pl.ds(rbase + r * CH, CH)])
        plsc.subcore_barrier()

        @pl.when(c == FAST_C)
        def _():
            blk0 = s * PAIR_BLKS
            pltpu.sync_copy(pck_hbm.at[pl.ds(blk0, FAST_BLKS)], pckb_v)
            _seg_pipeline(g_hbm, acc_sh, pckb_v, uidx_v, bufs_v,
                          gsems, ssems, FAST_BLKS, NBUF)

        @pl.when(c != FAST_C)
        def _():
            blk0 = s * PAIR_BLKS + FAST_BLKS
            pltpu.sync_copy(pck_hbm.at[pl.ds(blk0, SLOW_BLKS)],
                            pckb_v.at[pl.ds(0, SLOW_BLKS)])
            _seg_pipeline(g_hbm, acc_sh, pckb_v, uidx_v, bufs_v,
                          gsems, ssems, SLOW_BLKS, NBUF)

        plsc.subcore_barrier()
        pltpu.sync_copy(acc_sh.at[pl.ds(rbase, ROWS_PER_TILE)],
                        out_hbm.at[c, pl.ds(rbase, ROWS_PER_TILE)])

    return _sc_segsum


_sc_segsum64 = _make_sc_segsum(H)


LW = 16
EDGES_PER_T3 = EP // NS
NCHUNK3 = EDGES_PER_T3 // CH


@functools.partial(
    pl.kernel,
    out_type=jax.ShapeDtypeStruct((NP, LW), jnp.float32),
    mesh=_mesh,
    compiler_params=_sc_params,
    scratch_types=[
        pltpu.VMEM_SHARED((NP, LW), jnp.float32),
        pltpu.VMEM((NCHUNK3, CH), jnp.int32),
        pltpu.VMEM((LBUF, 2, CH), jnp.int32),
        pltpu.VMEM((LBUF, CH, LW), jnp.float32),
        pltpu.VMEM((ROWS_PER_TILE, LW), jnp.float32),
        pltpu.VMEM((ROWS_PER_TILE, LW), jnp.float32),
        pltpu.VMEM((ROWS_PER_TILE,), jnp.float32),
        pltpu.VMEM((CH,), jnp.int32),
        pltpu.SemaphoreType.DMA((LBUF,)),
        pltpu.SemaphoreType.DMA((LBUF,)),
    ],
)
def _sc_leaf(g3_hbm, pck_hbm, sb3_hbm, norm_hbm, fidx_hbm,
             ztile_hbm, out_hbm, acc_sh, pckb_v, uidx_v, bufs_v, a3_v, sb_v,
             norm_v, fc_v, gsem_a, ssem_a):
    gsems = [gsem_a.at[b] for b in range(LBUF)]
    ssems = [ssem_a.at[b] for b in range(LBUF)]
    c = lax.axis_index("c")
    s = lax.axis_index("s")

    @pl.when(c == FAST_C)
    def _():
        rbase = s * ROWS_PER_TILE
        pltpu.sync_copy(ztile_hbm, bufs_v.at[0])
        for r in range(ROWS_PER_TILE // CH):
            pltpu.sync_copy(bufs_v.at[0], acc_sh.at[pl.ds(rbase + r * CH, CH)])
        pltpu.sync_copy(pck_hbm.at[pl.ds(s * NCHUNK3, NCHUNK3)], pckb_v)
        plsc.subcore_barrier()
        _seg_pipeline(g3_hbm, acc_sh, pckb_v, uidx_v, bufs_v,
                      gsems, ssems, NCHUNK3, LBUF)
        plsc.subcore_barrier()

        pltpu.sync_copy(acc_sh.at[pl.ds(rbase, ROWS_PER_TILE)], a3_v)
        pltpu.sync_copy(sb3_hbm.at[pl.ds(rbase, ROWS_PER_TILE)], sb_v)
        pltpu.sync_copy(norm_hbm.at[pl.ds(rbase, ROWS_PER_TILE)], norm_v)
        for gi in range(ROWS_PER_TILE // 16):
            rid = lax.iota(jnp.int32, 16) + gi * 16
            nv = norm_v[pl.ds(gi * 16, 16)]
            for j in range(2):
                jv = jnp.full((16,), j, jnp.int32)
                av = plsc.load_gather(a3_v, [rid, jv])
                sv = plsc.load_gather(sb_v, [rid, jv])
                plsc.store_scatter(a3_v, [rid, jv], nv * av + sv)
        for ci in range(ROWS_PER_TILE // CH):
            pltpu.sync_copy(fidx_hbm.at[pl.ds(rbase + ci * CH, CH)], fc_v)
            pltpu.async_copy(a3_v.at[pl.ds(ci * CH, CH)],
                             out_hbm.at[fc_v], gsems[0]).wait()


R = 1000


def _tc1_body(d_ref, x_ref, w_ref, h1_ref, g1_ref, n_ref):
    deg = d_ref[:, 0:1] + d_ref[:, 1:2] + 1.0
    norm = lax.rsqrt(deg)
    h1 = jnp.dot(x_ref[...], w_ref[...], preferred_element_type=jnp.float32)
    h1_ref[...] = h1
    g1_ref[...] = norm * h1
    n_ref[...] = norm


def _tc2_body(a_ref, h1_ref, n_ref, b_ref, w_ref, t2_ref, g2_ref):
    a = a_ref[0] + a_ref[1]
    n = n_ref[...]
    pre = n * a + (n * n) * h1_ref[...] + b_ref[...]
    t2 = jnp.dot(pre, w_ref[...], preferred_element_type=jnp.float32)
    t2_ref[...] = t2
    g2_ref[...] = n * t2


def _tc3_body(a_ref, t2_ref, n_ref, bb_ref, wl_ref, bl_ref, we_ref, be_ref,
              g3_ref, sb3_ref, ye_ref, pool_ref):
    i = pl.program_id(0)
    a = a_ref[0] + a_ref[1]
    n = n_ref[...]
    pre = n * a + (n * n) * t2_ref[...] + bb_ref[...]
    h2 = jnp.where(pre >= 0, pre, 0.01 * pre)
    t3 = jnp.dot(h2, wl_ref[...], preferred_element_type=jnp.float32)
    g3_ref[...] = n * t3
    sb3_ref[...] = (n * n) * t3 + bl_ref[...]
    rows = lax.broadcasted_iota(jnp.int32, (R, 1), 0) + i * R
    gid = rows // MAXN
    valid = rows < N
    gcols = lax.broadcasted_iota(jnp.int32, (R, B), 1)
    onehot = jnp.where((gid == gcols) & valid, 1.0, 0.0)
    pp = lax.dot_general(onehot, h2, (((0,), (0,)), ((), ())),
                         preferred_element_type=jnp.float32)

    @pl.when(i == 0)
    def _():
        pool_ref[...] = pp

    @pl.when(i > 0)
    def _():
        pool_ref[...] = pool_ref[...] + pp

    @pl.when(i == NP // R - 1)
    def _():
        ye_ref[...] = jnp.dot(pool_ref[...], we_ref[...],
                              preferred_element_type=jnp.float32) + be_ref[...]


def _row_spec(width):
    return pl.BlockSpec((R, width), lambda i: (i, 0))


def _part_spec(width):
    return pl.BlockSpec((NC, R, width), lambda i: (0, i, 0))


def _full_spec(shape):
    return pl.BlockSpec(shape, lambda i: tuple(0 for _ in shape))


_tc1 = pl.pallas_call(
    _tc1_body,
    grid=(NP // R,),
    in_specs=[_row_spec(2), _row_spec(D), _full_spec((D, H))],
    out_specs=[_row_spec(H), _row_spec(H), _row_spec(1)],
    out_shape=[jax.ShapeDtypeStruct((NP, H), jnp.float32),
               jax.ShapeDtypeStruct((NP, H), jnp.float32),
               jax.ShapeDtypeStruct((NP, 1), jnp.float32)],
)

_tc2 = pl.pallas_call(
    _tc2_body,
    grid=(NP // R,),
    in_specs=[_part_spec(H), _row_spec(H), _row_spec(1),
              _full_spec((1, H)), _full_spec((H, H))],
    out_specs=[_row_spec(H), _row_spec(H)],
    out_shape=[jax.ShapeDtypeStruct((NP, H), jnp.float32),
               jax.ShapeDtypeStruct((NP, H), jnp.float32)],
)

_tc3 = pl.pallas_call(
    _tc3_body,
    grid=(NP // R,),
    in_specs=[_part_spec(H), _row_spec(H), _row_spec(1),
              _full_spec((1, H)), _full_spec((H, LW)), _full_spec((1, LW)),
              _full_spec((H, 1)), _full_spec((1, 1))],
    out_specs=[_row_spec(LW), _row_spec(LW),
               pl.BlockSpec((B, 1), lambda i: (0, 0)),
               pl.BlockSpec((B, H), lambda i: (0, 0))],
    out_shape=[jax.ShapeDtypeStruct((NP, LW), jnp.float32),
               jax.ShapeDtypeStruct((NP, LW), jnp.float32),
               jax.ShapeDtypeStruct((B, 1), jnp.float32),
               jax.ShapeDtypeStruct((B, H), jnp.float32)],
)


@jax.jit
def kernel(x, edge_index, k, batch, W_bb, b_bb, W_body, b_body,
           W_leaf, b_leaf, W_eos, b_eos):
    f32 = jnp.float32
    src2 = jnp.concatenate(
        [edge_index[0], jnp.full((EP - E,), N, jnp.int32)]).reshape(-1, CH)
    dst2 = jnp.concatenate(
        [edge_index[1], jnp.full((EP - E,), N, jnp.int32)]).reshape(-1, CH)
    pck = (dst2 << 16) | src2
    flat_idx = batch.astype(jnp.int32) * MAXN + k.astype(jnp.int32)
    fidx = jnp.concatenate([flat_idx, jnp.full((NP - N,), N, jnp.int32)])
    wl_p = jnp.pad(W_leaf, ((0, 0), (0, LW - 2)))
    bl_p = jnp.pad(b_leaf, ((0, LW - 2),)).reshape(1, LW)
    zrow = jnp.zeros((ROWS_PER_TILE,), f32)
    ztile64 = jnp.zeros((CH, H), f32)
    ztile16 = jnp.zeros((CH, LW), f32)

    degp = _sc_deg(dst2, zrow)
    deg2 = jnp.transpose(degp)
    h1, g1, norm = _tc1(deg2, x, W_bb)
    a1 = _sc_segsum64(g1, pck, ztile64)
    t2, g2 = _tc2(a1, h1, norm, b_bb.reshape(1, H), W_body)
    a2 = _sc_segsum64(g2, pck, ztile64)
    g3, sb3, ye, _pool = _tc3(a2, t2, norm, b_body.reshape(1, H), wl_p, bl_p,
                              W_eos, b_eos.reshape(1, 1))
    y16 = _sc_leaf(g3, pck, sb3, norm.reshape(NP), fidx, ztile16)
    y_leaf = y16[:N, :2].reshape(B, MAXN * 2)
    y_eos = ye.reshape(B)
    return (y_leaf, y_eos)
